# Initial kernel scaffold; baseline (speedup 1.0000x reference)
import functools

import jax
import jax.numpy as jnp
from jax import lax
from jax.experimental import pallas as pl
from jax.experimental.pallas import tpu as pltpu
from jax.experimental.pallas import tpu_sc as plsc

NT = 32
R = 313
RT = R + 1
NPADR = NT * R
DEGW = 320
CH = 4000
FLUSH = 2048
C = 256
CG = 128
FW = 64


def _wid():
    return lax.axis_index("s") * 2 + lax.axis_index("c")


@functools.cache
def _bucket_kernel(E):
    EPAD = E + C
    mesh = plsc.VectorSubcoreMesh(core_axis_name="c", subcore_axis_name="s")
    out_type = (
        jax.ShapeDtypeStruct((NT, EPAD), jnp.int32),
        jax.ShapeDtypeStruct((NT, EPAD), jnp.int32),
        jax.ShapeDtypeStruct((NT, 16), jnp.int32),
        jax.ShapeDtypeStruct((NT, DEGW), jnp.float32),
    )
    scratch = [
        pltpu.VMEM((CH,), jnp.int32),
        pltpu.VMEM((CH,), jnp.int32),
        pltpu.VMEM((FLUSH + 2 * C,), jnp.int32),
        pltpu.VMEM((FLUSH + 2 * C,), jnp.int32),
        pltpu.VMEM((DEGW,), jnp.float32),
        pltpu.VMEM((16,), jnp.int32),
    ]

    @functools.partial(pl.kernel, out_type=out_type, mesh=mesh,
                       scratch_types=scratch)
    def bucket(edge_h, srcl_h, dstl_h, cnt_h, deg_h,
               dbuf, sbuf, csrc, cdst, degv, cntv):
        wid = _wid()
        lo = wid * R

        def zb(i, carry):
            degv[pl.ds(i * 16, 16)] = jnp.zeros((16,), jnp.float32)
            return carry
        lax.fori_loop(0, DEGW // 16, zb, 0)

        ones = jnp.ones((16,), jnp.float32)

        def inner(j, carry):
            w, base = carry
            d = dbuf[pl.ds(j * 16, 16)]
            s = sbuf[pl.ds(j * 16, 16)]
            dl = d - lo
            m = (dl >= 0) & (dl < R)
            dlc = jnp.where(m, dl, R)
            plsc.addupdate_scatter(degv, [dlc], ones, mask=m)
            plsc.store_compressed(csrc.at[pl.ds(w, 16)], s, mask=m)
            plsc.store_compressed(cdst.at[pl.ds(w, 16)], dlc, mask=m)
            w = w + jnp.max(plsc.all_reduce_population_count(m))
            fl = w >= FLUSH

            @pl.when(fl)
            def _():
                pltpu.sync_copy(csrc.at[pl.ds(0, FLUSH)],
                                srcl_h.at[wid, pl.ds(base, FLUSH)])
                pltpu.sync_copy(cdst.at[pl.ds(0, FLUSH)],
                                dstl_h.at[wid, pl.ds(base, FLUSH)])
                rs = csrc[pl.ds(FLUSH, 16)]
                csrc[pl.ds(0, 16)] = rs
                rd = cdst[pl.ds(FLUSH, 16)]
                cdst[pl.ds(0, 16)] = rd

            w = jnp.where(fl, w - FLUSH, w)
            base = jnp.where(fl, base + FLUSH, base)
            return (w, base)

        def outer(i, carry):
            pltpu.sync_copy(edge_h.at[1, pl.ds(i * CH, CH)], dbuf)
            pltpu.sync_copy(edge_h.at[0, pl.ds(i * CH, CH)], sbuf)
            return lax.fori_loop(0, CH // 16, inner, carry)

        w, base = lax.fori_loop(0, E // CH, outer,
                                (jnp.int32(0), jnp.int32(0)))

        wp = ((w + (C - 1)) // C) * C

        def padb(k, carry):
            csrc[pl.ds(w + k * 16, 16)] = jnp.zeros((16,), jnp.int32)
            cdst[pl.ds(w + k * 16, 16)] = jnp.full((16,), R, jnp.int32)
            return carry
        lax.fori_loop(0, (wp - w + 15) // 16, padb, 0)

        def drain(k, carry):
            pltpu.sync_copy(csrc.at[pl.ds(k * C, C)],
                            srcl_h.at[wid, pl.ds(base + k * C, C)])
            pltpu.sync_copy(cdst.at[pl.ds(k * C, C)],
                            dstl_h.at[wid, pl.ds(base + k * C, C)])
            return carry
        lax.fori_loop(0, wp // C, drain, 0)

        cntv[...] = jnp.full((16,), base + wp, jnp.int32)
        pltpu.sync_copy(cntv, cnt_h.at[wid])
        pltpu.sync_copy(degv, deg_h.at[wid])

    return bucket


@functools.cache
def _acc_kernel(E, npass):
    ACC = RT * FW
    mesh = plsc.VectorSubcoreMesh(core_axis_name="c", subcore_axis_name="s")
    out_type = tuple(
        jax.ShapeDtypeStruct((npass, NPADR * FW), jnp.float32)
        for _ in range(4))
    scratch = [
        pltpu.VMEM((ACC,), jnp.float32),
        pltpu.VMEM((ACC,), jnp.float32),
        pltpu.VMEM((ACC,), jnp.float32),
        pltpu.VMEM((ACC,), jnp.float32),
        pltpu.VMEM((C, FW), jnp.float32),
        pltpu.VMEM((C,), jnp.int32),
        pltpu.VMEM((2, CG), jnp.int32),
        pltpu.VMEM((16,), jnp.int32),
        pltpu.SMEM((C,), jnp.int32),
        pltpu.SemaphoreType.DMA,
        pltpu.SemaphoreType.DMA,
    ]

    @functools.partial(pl.kernel, out_type=out_type, mesh=mesh,
                       scratch_types=scratch)
    def acc(table_h, srcl_h, dstl_h, cnt_h, s1_h, s2_h, mn_h, mx_h,
            accS, accQ, accMn, accMx, rows, sbufv, gi, cntv, dsm,
            sem0, sem1):
        wid = _wid()
        lo = wid * R
        pltpu.sync_copy(cnt_h.at[wid], cntv)
        cnt = jnp.max(cntv[...])
        nch = cnt // C

        for f in range(npass):
            def zb(i, carry):
                z = jnp.zeros((16,), jnp.float32)
                accS[pl.ds(i * 16, 16)] = z
                accQ[pl.ds(i * 16, 16)] = z
                accMn[pl.ds(i * 16, 16)] = jnp.full((16,), 3e38, jnp.float32)
                accMx[pl.ds(i * 16, 16)] = jnp.full((16,), -3e38,
                                                    jnp.float32)
                return carry
            lax.fori_loop(0, ACC // 16, zb, 0)

            def chunk(ch, carry):
                eb = ch * C
                pltpu.sync_copy(srcl_h.at[wid, pl.ds(eb, C)], sbufv)
                pltpu.sync_copy(dstl_h.at[wid, pl.ds(eb, C)], dsm)
                for k in range(C // 16):
                    v = sbufv[pl.ds(k * 16, 16)]
                    gi[k * 16 // CG, pl.ds((k * 16) % CG, 16)] = \
                        v * npass + f
                d0 = pltpu.async_copy(table_h.at[gi.at[0]],
                                      rows.at[pl.ds(0, CG)], sem0)
                d1 = pltpu.async_copy(table_h.at[gi.at[1]],
                                      rows.at[pl.ds(CG, CG)], sem1)
                d0.wait()
                d1.wait()

                def edge(e, carry2):
                    off = dsm[e] * FW
                    for k in range(FW // 16):
                        r = rows[e, pl.ds(k * 16, 16)]
                        plsc.addupdate(accS.at[pl.ds(off + k * 16, 16)], r)
                        plsc.addupdate(accQ.at[pl.ds(off + k * 16, 16)],
                                       r * r)
                        cm = accMn[pl.ds(off + k * 16, 16)]
                        accMn[pl.ds(off + k * 16, 16)] = jnp.minimum(cm, r)
                        cx = accMx[pl.ds(off + k * 16, 16)]
                        accMx[pl.ds(off + k * 16, 16)] = jnp.maximum(cx, r)
                    return carry2
                lax.fori_loop(0, C, edge, 0)
                return carry
            lax.fori_loop(0, nch, chunk, 0)

            pltpu.sync_copy(accS.at[pl.ds(0, R * FW)],
                            s1_h.at[f, pl.ds(lo * FW, R * FW)])
            pltpu.sync_copy(accQ.at[pl.ds(0, R * FW)],
                            s2_h.at[f, pl.ds(lo * FW, R * FW)])
            pltpu.sync_copy(accMn.at[pl.ds(0, R * FW)],
                            mn_h.at[f, pl.ds(lo * FW, R * FW)])
            pltpu.sync_copy(accMx.at[pl.ds(0, R * FW)],
                            mx_h.at[f, pl.ds(lo * FW, R * FW)])

    return acc


def _tc_pre(x, Wd, Ws, b, blk=2000):
    n, fin = x.shape
    fo = Wd.shape[1]

    def body(xr, wdr, wsr, br, ar, btr):
        xb = xr[...]
        ar[...] = jnp.dot(xb, wdr[...],
                          preferred_element_type=jnp.float32) + br[...]
        btr[...] = jnp.dot(xb, wsr[...], preferred_element_type=jnp.float32)

    return pl.pallas_call(
        body,
        grid=(n // blk,),
        in_specs=[
            pl.BlockSpec((blk, fin), lambda i: (i, 0)),
            pl.BlockSpec((fin, fo), lambda i: (0, 0)),
            pl.BlockSpec((fin, fo), lambda i: (0, 0)),
            pl.BlockSpec((1, fo), lambda i: (0, 0)),
        ],
        out_specs=[
            pl.BlockSpec((blk, fo), lambda i: (i, 0)),
            pl.BlockSpec((blk, fo), lambda i: (i, 0)),
        ],
        out_shape=[jax.ShapeDtypeStruct((n, fo), jnp.float32)] * 2,
    )(x, Wd, Ws, b.reshape(1, fo))


def _tc_avglog(degp, n):
    def body(degr, outr):
        d = degr[...]
        col = lax.broadcasted_iota(jnp.int32, (NT, DEGW), 1)
        row = lax.broadcasted_iota(jnp.int32, (NT, DEGW), 0)
        valid = (col < R) & (row * R + col < n)
        outr[0, 0] = jnp.sum(jnp.where(valid, jnp.log(d + 1.0), 0.0)) / n

    return pl.pallas_call(
        body,
        in_specs=[pl.BlockSpec((NT, DEGW), lambda: (0, 0))],
        out_specs=pl.BlockSpec(memory_space=pltpu.SMEM),
        out_shape=jax.ShapeDtypeStruct((1, 1), jnp.float32),
    )(degp)


def _tc_post(xin, A, s1, s2, mn, mx, deg, avgl, postW, postb, linW, linb,
             avg_lin, pre_next=None, blk=1000):
    n, fin = xin.shape
    f = A.shape[1]
    npass = s1.shape[0]
    hid = postW.shape[1]
    ho = linW.shape[1]
    wx = postW[:f]
    w1 = postW[f:5 * f]
    w2 = postW[5 * f:9 * f]
    w3 = postW[9 * f:13 * f]
    w4 = postW[13 * f:17 * f]
    fused = pre_next is not None
    if fused:
        pnW, pnb = pre_next
        fn = pnW.shape[1]

    def body(xr, ar, s1r, s2r, mnr, mxr, degr, avr,
             wxr, w1r, w2r, w3r, w4r, pbr, lwr, lbr, *rest):
        deg_b = degr[...]
        dc = jnp.maximum(deg_b, 1.0)
        has = deg_b > 0.0
        cA = ar[...]
        if npass == 2:
            s1v = jnp.concatenate([s1r[0], s1r[1]], axis=-1)
            s2v = jnp.concatenate([s2r[0], s2r[1]], axis=-1)
            mnv = jnp.concatenate([mnr[0], mnr[1]], axis=-1)
            mxv = jnp.concatenate([mxr[0], mxr[1]], axis=-1)
        else:
            s1v, s2v, mnv, mxv = s1r[0], s2r[0], mnr[0], mxr[0]
        s1d = s1v / dc
        mean = jnp.where(has, cA + s1d, 0.0)
        mnx = jnp.where(has, cA + mnv, 0.0)
        mxx = jnp.where(has, cA + mxv, 0.0)
        var = jnp.maximum(s2v / dc - s1d * s1d, 0.0)
        std = jnp.sqrt(var + 1e-5)
        aggr = jnp.concatenate([mean, mnx, mxx, std], axis=-1)
        al = avr[0, 0]
        ld = jnp.log(dc + 1.0)
        o = jnp.dot(xr[...], wxr[...], preferred_element_type=jnp.float32)
        o += jnp.dot(aggr, w1r[...], preferred_element_type=jnp.float32)
        o += jnp.dot(aggr * (ld / al), w2r[...],
                     preferred_element_type=jnp.float32)
        o += jnp.dot(aggr * (al / ld), w3r[...],
                     preferred_element_type=jnp.float32)
        o += jnp.dot(aggr * (dc / avg_lin), w4r[...],
                     preferred_element_type=jnp.float32)
        o += pbr[...]
        o = jnp.dot(o, lwr[...], preferred_element_type=jnp.float32)
        o += lbr[...]
        if fused:
            wdr, wsr, pnbr, hr, a2r, b2r = rest
            h = jnp.maximum(o, 0.0)
            hr[...] = h
            a2r[...] = jnp.dot(h, wdr[...],
                               preferred_element_type=jnp.float32) + pnbr[...]
            b2r[...] = jnp.dot(h, wsr[...],
                               preferred_element_type=jnp.float32)
        else:
            outr, = rest
            om = o - jnp.max(o, axis=1, keepdims=True)
            outr[...] = om - jnp.log(
                jnp.sum(jnp.exp(om), axis=1, keepdims=True))

    def full(a):
        return pl.BlockSpec(a.shape, lambda i: (0,) * a.ndim)

    in_specs = [
        pl.BlockSpec((blk, fin), lambda i: (i, 0)),
        pl.BlockSpec((blk, f), lambda i: (i, 0)),
    ] + [pl.BlockSpec((npass, blk, FW), lambda i: (0, i, 0))] * 4 + [
        pl.BlockSpec((blk, 1), lambda i: (i, 0)),
        pl.BlockSpec(memory_space=pltpu.SMEM),
        full(wx), full(w1), full(w2), full(w3), full(w4),
        pl.BlockSpec((1, hid), lambda i: (0, 0)),
        full(linW),
        pl.BlockSpec((1, ho), lambda i: (0, 0)),
    ]
    args = [xin, A, s1, s2, mn, mx, deg, avgl,
            wx, w1, w2, w3, w4, postb.reshape(1, hid), linW,
            linb.reshape(1, ho)]
    if fused:
        in_specs += [full(pnW[:hid]), full(pnW[hid:]),
                     pl.BlockSpec((1, fn), lambda i: (0, 0))]
        args += [pnW[:hid], pnW[hid:], pnb.reshape(1, fn)]
        out_specs = [pl.BlockSpec((blk, ho), lambda i: (i, 0)),
                     pl.BlockSpec((blk, fn), lambda i: (i, 0)),
                     pl.BlockSpec((blk, fn), lambda i: (i, 0))]
        out_shape = [jax.ShapeDtypeStruct((n, ho), jnp.float32),
                     jax.ShapeDtypeStruct((n, fn), jnp.float32),
                     jax.ShapeDtypeStruct((n, fn), jnp.float32)]
    else:
        out_specs = [pl.BlockSpec((blk, ho), lambda i: (i, 0))]
        out_shape = [jax.ShapeDtypeStruct((n, ho), jnp.float32)]

    return pl.pallas_call(
        body,
        grid=(n // blk,),
        in_specs=in_specs,
        out_specs=out_specs,
        out_shape=out_shape,
    )(*args)


def kernel(x, edge_index, pre1_W, pre1_b, post1_W, post1_b, lin1_W, lin1_b,
           pre2_W, pre2_b, post2_W, post2_b, lin2_W, lin2_b):
    n, f_in = x.shape
    e = edge_index.shape[1]
    f1 = pre1_W.shape[1]
    avg_lin = float(e) / float(n)

    srcl, dstl, cnth, degp = _bucket_kernel(e)(edge_index)
    deg = degp[:, :R].reshape(-1)[:n].reshape(n, 1)
    avgl = _tc_avglog(degp, n)

    a1, b1 = _tc_pre(x, pre1_W[:f_in], pre1_W[f_in:], pre1_b)
    np1 = f1 // FW
    table1 = b1.reshape(np1 * n, FW)
    s1, s2, mn, mx = _acc_kernel(e, np1)(table1, srcl, dstl, cnth)

    def rs1(t):
        return t.reshape(np1, NPADR, FW)

    h, a2, b2 = _tc_post(x, a1, rs1(s1), rs1(s2), rs1(mn), rs1(mx), deg,
                         avgl, post1_W, post1_b, lin1_W, lin1_b, avg_lin,
                         pre_next=(pre2_W, pre2_b))

    f2 = pre2_W.shape[1]
    np2 = f2 // FW
    s1b, s2b, mnb, mxb = _acc_kernel(e, np2)(b2, srcl, dstl, cnth)

    def rs2(t):
        return t.reshape(np2, NPADR, FW)

    (out,) = _tc_post(h, a2, rs2(s1b), rs2(s2b), rs2(mnb), rs2(mxb), deg,
                      avgl, post2_W, post2_b, lin2_W, lin2_b, avg_lin)
    return out


# speedup vs baseline: 3.7949x; 3.7949x over previous
import functools

import jax
import jax.numpy as jnp
from jax import lax
from jax.experimental import pallas as pl
from jax.experimental.pallas import tpu as pltpu
from jax.experimental.pallas import tpu_sc as plsc

NT = 32
R = 313
RT = R + 1
NPADR = NT * R
DEGW = 320
CH = 4000
FLUSH = 2048
C = 256
CG = 128
FW = 64


def _wid():
    return lax.axis_index("s") * 2 + lax.axis_index("c")


@functools.cache
def _bucket_kernel(E):
    EPAD = E + C
    mesh = plsc.VectorSubcoreMesh(core_axis_name="c", subcore_axis_name="s")
    out_type = (
        jax.ShapeDtypeStruct((NT, EPAD), jnp.int32),
        jax.ShapeDtypeStruct((NT, EPAD), jnp.int32),
        jax.ShapeDtypeStruct((NT, 16), jnp.int32),
        jax.ShapeDtypeStruct((NT, DEGW), jnp.float32),
    )
    scratch = [
        pltpu.VMEM((CH,), jnp.int32),
        pltpu.VMEM((CH,), jnp.int32),
        pltpu.VMEM((FLUSH + 2 * C,), jnp.int32),
        pltpu.VMEM((FLUSH + 2 * C,), jnp.int32),
        pltpu.VMEM((DEGW,), jnp.float32),
        pltpu.VMEM((16,), jnp.int32),
    ]

    @functools.partial(pl.kernel, out_type=out_type, mesh=mesh,
                       scratch_types=scratch,
                       compiler_params=pltpu.CompilerParams(
                           use_tc_tiling_on_sc=False,
                           needs_layout_passes=False))
    def bucket(src_h, dst_h, srcl_h, dstl_h, cnt_h, deg_h,
               dbuf, sbuf, csrc, cdst, degv, cntv):
        wid = _wid()
        lo = wid * R

        def zb(i, carry):
            degv[pl.ds(i * 16, 16)] = jnp.zeros((16,), jnp.float32)
            return carry
        lax.fori_loop(0, DEGW // 16, zb, 0)

        ones = jnp.ones((16,), jnp.float32)

        def inner(j, carry):
            w, base = carry
            d = dbuf[pl.ds(j * 16, 16)]
            s = sbuf[pl.ds(j * 16, 16)]
            dl = d - lo
            m = (dl >= 0) & (dl < R)
            dlc = jnp.where(m, dl, R)
            plsc.addupdate_scatter(degv, [dlc], ones, mask=m)
            plsc.store_compressed(csrc.at[pl.ds(w, 16)], s, mask=m)
            plsc.store_compressed(cdst.at[pl.ds(w, 16)], dlc, mask=m)
            w = w + jnp.max(plsc.all_reduce_population_count(m))
            fl = w >= FLUSH

            @pl.when(fl)
            def _():
                fb = pl.multiple_of(base, FLUSH)
                pltpu.sync_copy(csrc.at[pl.ds(0, FLUSH)],
                                srcl_h.at[wid, pl.ds(fb, FLUSH)])
                pltpu.sync_copy(cdst.at[pl.ds(0, FLUSH)],
                                dstl_h.at[wid, pl.ds(fb, FLUSH)])
                rs = csrc[pl.ds(FLUSH, 16)]
                csrc[pl.ds(0, 16)] = rs
                rd = cdst[pl.ds(FLUSH, 16)]
                cdst[pl.ds(0, 16)] = rd

            w = jnp.where(fl, w - FLUSH, w)
            base = jnp.where(fl, base + FLUSH, base)
            return (w, base)

        def outer(i, carry):
            ib = pl.multiple_of(i * CH, 8)
            pltpu.sync_copy(dst_h.at[pl.ds(ib, CH)], dbuf)
            pltpu.sync_copy(src_h.at[pl.ds(ib, CH)], sbuf)
            return lax.fori_loop(0, CH // 16, inner, carry)

        w, base = lax.fori_loop(0, E // CH, outer,
                                (jnp.int32(0), jnp.int32(0)))

        wp = ((w + (C - 1)) // C) * C

        def padb(k, carry):
            csrc[pl.ds(w + k * 16, 16)] = jnp.zeros((16,), jnp.int32)
            cdst[pl.ds(w + k * 16, 16)] = jnp.full((16,), R, jnp.int32)
            return carry
        lax.fori_loop(0, (wp - w + 15) // 16, padb, 0)

        def drain(k, carry):
            db = pl.multiple_of(base + k * C, C)
            pltpu.sync_copy(csrc.at[pl.ds(k * C, C)],
                            srcl_h.at[wid, pl.ds(db, C)])
            pltpu.sync_copy(cdst.at[pl.ds(k * C, C)],
                            dstl_h.at[wid, pl.ds(db, C)])
            return carry
        lax.fori_loop(0, wp // C, drain, 0)

        cntv[...] = jnp.full((16,), base + wp, jnp.int32)
        pltpu.sync_copy(cntv, cnt_h.at[wid])
        pltpu.sync_copy(degv, deg_h.at[wid])

    return bucket


@functools.cache
def _acc_kernel(E, npass):
    ACC = RT * FW
    mesh = plsc.VectorSubcoreMesh(core_axis_name="c", subcore_axis_name="s")
    out_type = tuple(
        jax.ShapeDtypeStruct((npass, NPADR * FW), jnp.float32)
        for _ in range(4))
    scratch = [
        pltpu.VMEM((ACC,), jnp.float32),
        pltpu.VMEM((ACC,), jnp.float32),
        pltpu.VMEM((ACC,), jnp.float32),
        pltpu.VMEM((ACC,), jnp.float32),
        pltpu.VMEM((C, FW), jnp.float32),
        pltpu.VMEM((C,), jnp.int32),
        pltpu.VMEM((C,), jnp.int32),
        pltpu.VMEM((2, CG), jnp.int32),
        pltpu.VMEM((16,), jnp.int32),
        pltpu.SemaphoreType.DMA,
        pltpu.SemaphoreType.DMA,
    ]

    @functools.partial(pl.kernel, out_type=out_type, mesh=mesh,
                       scratch_types=scratch,
                       compiler_params=pltpu.CompilerParams(
                           use_tc_tiling_on_sc=False,
                           needs_layout_passes=False))
    def acc(table_h, srcl_h, dstl_h, cnt_h, s1_h, s2_h, mn_h, mx_h,
            accS, accQ, accMn, accMx, rows, sbufv, dbufv, gi, cntv,
            sem0, sem1):
        wid = _wid()
        lo = wid * R
        pltpu.sync_copy(cnt_h.at[wid], cntv)
        cnt = jnp.max(cntv[...])
        nch = cnt // C

        for f in range(npass):
            def zb(i, carry):
                z = jnp.zeros((16,), jnp.float32)
                accS[pl.ds(i * 16, 16)] = z
                accQ[pl.ds(i * 16, 16)] = z
                accMn[pl.ds(i * 16, 16)] = jnp.full((16,), 3e38, jnp.float32)
                accMx[pl.ds(i * 16, 16)] = jnp.full((16,), -3e38,
                                                    jnp.float32)
                return carry
            lax.fori_loop(0, ACC // 16, zb, 0)

            def chunk(ch, carry):
                eb = pl.multiple_of(ch * C, C)
                pltpu.sync_copy(srcl_h.at[wid, pl.ds(eb, C)], sbufv)
                pltpu.sync_copy(dstl_h.at[wid, pl.ds(eb, C)], dbufv)
                for k in range(C // 16):
                    v = sbufv[pl.ds(k * 16, 16)]
                    gi[k * 16 // CG, pl.ds((k * 16) % CG, 16)] = \
                        v * npass + f
                d0 = pltpu.async_copy(table_h.at[gi.at[0]],
                                      rows.at[pl.ds(0, CG)], sem0)
                d1 = pltpu.async_copy(table_h.at[gi.at[1]],
                                      rows.at[pl.ds(CG, CG)], sem1)
                d0.wait()
                d1.wait()

                def edge(g, carry2):
                    dv = dbufv[pl.ds(g * 16, 16)] * FW
                    for l in range(16):
                        off = dv[l]
                        e = g * 16 + l
                        for k in range(FW // 16):
                            r = rows[e, pl.ds(k * 16, 16)]
                            plsc.addupdate(
                                accS.at[pl.ds(off + k * 16, 16)], r)
                            plsc.addupdate(
                                accQ.at[pl.ds(off + k * 16, 16)], r * r)
                            cm = accMn[pl.ds(off + k * 16, 16)]
                            accMn[pl.ds(off + k * 16, 16)] = \
                                jnp.minimum(cm, r)
                            cx = accMx[pl.ds(off + k * 16, 16)]
                            accMx[pl.ds(off + k * 16, 16)] = \
                                jnp.maximum(cx, r)
                    return carry2
                lax.fori_loop(0, C // 16, edge, 0)
                return carry
            lax.fori_loop(0, nch, chunk, 0)

            ob = pl.multiple_of(lo * FW, FW)
            pltpu.sync_copy(accS.at[pl.ds(0, R * FW)],
                            s1_h.at[f, pl.ds(ob, R * FW)])
            pltpu.sync_copy(accQ.at[pl.ds(0, R * FW)],
                            s2_h.at[f, pl.ds(ob, R * FW)])
            pltpu.sync_copy(accMn.at[pl.ds(0, R * FW)],
                            mn_h.at[f, pl.ds(ob, R * FW)])
            pltpu.sync_copy(accMx.at[pl.ds(0, R * FW)],
                            mx_h.at[f, pl.ds(ob, R * FW)])

    return acc


def _tc_pre(x, Wd, Ws, b, blk=2000):
    n, fin = x.shape
    fo = Wd.shape[1]

    def body(xr, wdr, wsr, br, ar, btr):
        xb = xr[...]
        ar[...] = jnp.dot(xb, wdr[...],
                          preferred_element_type=jnp.float32) + br[...]
        btr[...] = jnp.dot(xb, wsr[...], preferred_element_type=jnp.float32)

    return pl.pallas_call(
        body,
        grid=(n // blk,),
        in_specs=[
            pl.BlockSpec((blk, fin), lambda i: (i, 0)),
            pl.BlockSpec((fin, fo), lambda i: (0, 0)),
            pl.BlockSpec((fin, fo), lambda i: (0, 0)),
            pl.BlockSpec((1, fo), lambda i: (0, 0)),
        ],
        out_specs=[
            pl.BlockSpec((blk, fo), lambda i: (i, 0)),
            pl.BlockSpec((blk, fo), lambda i: (i, 0)),
        ],
        out_shape=[jax.ShapeDtypeStruct((n, fo), jnp.float32)] * 2,
    )(x, Wd, Ws, b.reshape(1, fo))


def _tc_avglog(degp, n):
    def body(degr, outr):
        d = degr[...]
        col = lax.broadcasted_iota(jnp.int32, (NT, DEGW), 1)
        row = lax.broadcasted_iota(jnp.int32, (NT, DEGW), 0)
        valid = (col < R) & (row * R + col < n)
        outr[0, 0] = jnp.sum(jnp.where(valid, jnp.log(d + 1.0), 0.0)) / n

    return pl.pallas_call(
        body,
        in_specs=[pl.BlockSpec((NT, DEGW), lambda: (0, 0))],
        out_specs=pl.BlockSpec(memory_space=pltpu.SMEM),
        out_shape=jax.ShapeDtypeStruct((1, 1), jnp.float32),
    )(degp)


def _tc_post(xin, A, s1, s2, mn, mx, deg, avgl, postW, postb, linW, linb,
             avg_lin, pre_next=None, blk=1000):
    n, fin = xin.shape
    f = A.shape[1]
    npass = s1.shape[0]
    hid = postW.shape[1]
    ho = linW.shape[1]
    wx = postW[:f]
    w1 = postW[f:5 * f]
    w2 = postW[5 * f:9 * f]
    w3 = postW[9 * f:13 * f]
    w4 = postW[13 * f:17 * f]
    fused = pre_next is not None
    if fused:
        pnW, pnb = pre_next
        fn = pnW.shape[1]

    def body(xr, ar, s1r, s2r, mnr, mxr, degr, avr,
             wxr, w1r, w2r, w3r, w4r, pbr, lwr, lbr, *rest):
        deg_b = degr[...]
        dc = jnp.maximum(deg_b, 1.0)
        has = deg_b > 0.0
        cA = ar[...]
        if npass == 2:
            s1v = jnp.concatenate([s1r[0], s1r[1]], axis=-1)
            s2v = jnp.concatenate([s2r[0], s2r[1]], axis=-1)
            mnv = jnp.concatenate([mnr[0], mnr[1]], axis=-1)
            mxv = jnp.concatenate([mxr[0], mxr[1]], axis=-1)
        else:
            s1v, s2v, mnv, mxv = s1r[0], s2r[0], mnr[0], mxr[0]
        s1d = s1v / dc
        mean = jnp.where(has, cA + s1d, 0.0)
        mnx = jnp.where(has, cA + mnv, 0.0)
        mxx = jnp.where(has, cA + mxv, 0.0)
        var = jnp.maximum(s2v / dc - s1d * s1d, 0.0)
        std = jnp.sqrt(var + 1e-5)
        aggr = jnp.concatenate([mean, mnx, mxx, std], axis=-1)
        al = avr[0, 0]
        ld = jnp.log(dc + 1.0)
        o = jnp.dot(xr[...], wxr[...], preferred_element_type=jnp.float32)
        o += jnp.dot(aggr, w1r[...], preferred_element_type=jnp.float32)
        o += jnp.dot(aggr * (ld / al), w2r[...],
                     preferred_element_type=jnp.float32)
        o += jnp.dot(aggr * (al / ld), w3r[...],
                     preferred_element_type=jnp.float32)
        o += jnp.dot(aggr * (dc / avg_lin), w4r[...],
                     preferred_element_type=jnp.float32)
        o += pbr[...]
        o = jnp.dot(o, lwr[...], preferred_element_type=jnp.float32)
        o += lbr[...]
        if fused:
            wdr, wsr, pnbr, hr, a2r, b2r = rest
            h = jnp.maximum(o, 0.0)
            hr[...] = h
            a2r[...] = jnp.dot(h, wdr[...],
                               preferred_element_type=jnp.float32) + pnbr[...]
            b2r[...] = jnp.dot(h, wsr[...],
                               preferred_element_type=jnp.float32)
        else:
            outr, = rest
            om = o - jnp.max(o, axis=1, keepdims=True)
            outr[...] = om - jnp.log(
                jnp.sum(jnp.exp(om), axis=1, keepdims=True))

    def full(a):
        return pl.BlockSpec(a.shape, lambda i: (0,) * a.ndim)

    in_specs = [
        pl.BlockSpec((blk, fin), lambda i: (i, 0)),
        pl.BlockSpec((blk, f), lambda i: (i, 0)),
    ] + [pl.BlockSpec((npass, blk, FW), lambda i: (0, i, 0))] * 4 + [
        pl.BlockSpec((blk, 1), lambda i: (i, 0)),
        pl.BlockSpec(memory_space=pltpu.SMEM),
        full(wx), full(w1), full(w2), full(w3), full(w4),
        pl.BlockSpec((1, hid), lambda i: (0, 0)),
        full(linW),
        pl.BlockSpec((1, ho), lambda i: (0, 0)),
    ]
    args = [xin, A, s1, s2, mn, mx, deg, avgl,
            wx, w1, w2, w3, w4, postb.reshape(1, hid), linW,
            linb.reshape(1, ho)]
    if fused:
        in_specs += [full(pnW[:hid]), full(pnW[hid:]),
                     pl.BlockSpec((1, fn), lambda i: (0, 0))]
        args += [pnW[:hid], pnW[hid:], pnb.reshape(1, fn)]
        out_specs = [pl.BlockSpec((blk, ho), lambda i: (i, 0)),
                     pl.BlockSpec((blk, fn), lambda i: (i, 0)),
                     pl.BlockSpec((blk, fn), lambda i: (i, 0))]
        out_shape = [jax.ShapeDtypeStruct((n, ho), jnp.float32),
                     jax.ShapeDtypeStruct((n, fn), jnp.float32),
                     jax.ShapeDtypeStruct((n, fn), jnp.float32)]
    else:
        out_specs = [pl.BlockSpec((blk, ho), lambda i: (i, 0))]
        out_shape = [jax.ShapeDtypeStruct((n, ho), jnp.float32)]

    return pl.pallas_call(
        body,
        grid=(n // blk,),
        in_specs=in_specs,
        out_specs=out_specs,
        out_shape=out_shape,
    )(*args)


def kernel(x, edge_index, pre1_W, pre1_b, post1_W, post1_b, lin1_W, lin1_b,
           pre2_W, pre2_b, post2_W, post2_b, lin2_W, lin2_b):
    n, f_in = x.shape
    e = edge_index.shape[1]
    f1 = pre1_W.shape[1]
    avg_lin = float(e) / float(n)

    srcl, dstl, cnth, degp = _bucket_kernel(e)(edge_index[0],
                                               edge_index[1])
    deg = degp[:, :R].reshape(-1)[:n].reshape(n, 1)
    avgl = _tc_avglog(degp, n)

    a1, b1 = _tc_pre(x, pre1_W[:f_in], pre1_W[f_in:], pre1_b)
    np1 = f1 // FW
    table1 = b1.reshape(np1 * n, FW)
    s1, s2, mn, mx = _acc_kernel(e, np1)(table1, srcl, dstl, cnth)

    def rs1(t):
        return t.reshape(np1, NPADR, FW)

    h, a2, b2 = _tc_post(x, a1, rs1(s1), rs1(s2), rs1(mn), rs1(mx), deg,
                         avgl, post1_W, post1_b, lin1_W, lin1_b, avg_lin,
                         pre_next=(pre2_W, pre2_b))

    f2 = pre2_W.shape[1]
    np2 = f2 // FW
    s1b, s2b, mnb, mxb = _acc_kernel(e, np2)(b2, srcl, dstl, cnth)

    def rs2(t):
        return t.reshape(np2, NPADR, FW)

    (out,) = _tc_post(h, a2, rs2(s1b), rs2(s2b), rs2(mnb), rs2(mxb), deg,
                      avgl, post2_W, post2_b, lin2_W, lin2_b, avg_lin)
    return out


# speedup vs baseline: 3.9592x; 1.0433x over previous
import functools

import jax
import jax.numpy as jnp
from jax import lax
from jax.experimental import pallas as pl
from jax.experimental.pallas import tpu as pltpu
from jax.experimental.pallas import tpu_sc as plsc

NT = 32
R = 313
RT = R + 1
NPADR = NT * R
DEGW = 320
CH = 4000
FLUSH = 2048
C = 256
CG = 128
FW = 64


def _wid():
    return lax.axis_index("s") * 2 + lax.axis_index("c")


@functools.cache
def _bucket_kernel(E):
    EPAD = E + C
    mesh = plsc.VectorSubcoreMesh(core_axis_name="c", subcore_axis_name="s")
    out_type = (
        jax.ShapeDtypeStruct((NT, EPAD), jnp.int32),
        jax.ShapeDtypeStruct((NT, EPAD), jnp.int32),
        jax.ShapeDtypeStruct((NT, 16), jnp.int32),
        jax.ShapeDtypeStruct((NT, DEGW), jnp.float32),
    )
    scratch = [
        pltpu.VMEM((CH,), jnp.int32),
        pltpu.VMEM((CH,), jnp.int32),
        pltpu.VMEM((CH,), jnp.int32),
        pltpu.VMEM((CH,), jnp.int32),
        pltpu.VMEM((FLUSH + 2 * C,), jnp.int32),
        pltpu.VMEM((FLUSH + 2 * C,), jnp.int32),
        pltpu.VMEM((DEGW,), jnp.float32),
        pltpu.VMEM((16,), jnp.int32),
        pltpu.SemaphoreType.DMA,
        pltpu.SemaphoreType.DMA,
    ]

    @functools.partial(pl.kernel, out_type=out_type, mesh=mesh,
                       scratch_types=scratch,
                       compiler_params=pltpu.CompilerParams(
                           use_tc_tiling_on_sc=False,
                           needs_layout_passes=False))
    def bucket(src_h, dst_h, srcl_h, dstl_h, cnt_h, deg_h,
               dbufA, sbufA, dbufB, sbufB, csrc, cdst, degv, cntv,
               semA, semB):
        wid = _wid()
        lo = wid * R

        def zb(i, carry):
            degv[pl.ds(i * 16, 16)] = jnp.zeros((16,), jnp.float32)
            return carry
        lax.fori_loop(0, DEGW // 16, zb, 0)

        ones = jnp.ones((16,), jnp.float32)

        def make_inner(dbufX, sbufX):
            def inner(j, carry):
                w, base = carry
                d = dbufX[pl.ds(j * 16, 16)]
                s = sbufX[pl.ds(j * 16, 16)]
                dl = d - lo
                m = (dl >= 0) & (dl < R)
                dlc = jnp.where(m, dl, R)
                plsc.addupdate_scatter(degv, [dlc], ones, mask=m)
                plsc.store_compressed(csrc.at[pl.ds(w, 16)], s, mask=m)
                plsc.store_compressed(cdst.at[pl.ds(w, 16)], dlc, mask=m)
                w = w + plsc.all_reduce_population_count(m)[0]
                fl = w >= FLUSH

                @pl.when(fl)
                def _():
                    fb = pl.multiple_of(base, FLUSH)
                    pltpu.sync_copy(csrc.at[pl.ds(0, FLUSH)],
                                    srcl_h.at[wid, pl.ds(fb, FLUSH)])
                    pltpu.sync_copy(cdst.at[pl.ds(0, FLUSH)],
                                    dstl_h.at[wid, pl.ds(fb, FLUSH)])
                    rs = csrc[pl.ds(FLUSH, 16)]
                    csrc[pl.ds(0, 16)] = rs
                    rd = cdst[pl.ds(FLUSH, 16)]
                    cdst[pl.ds(0, 16)] = rd

                w = jnp.where(fl, w - FLUSH, w)
                base = jnp.where(fl, base + FLUSH, base)
                return (w, base)
            return inner

        def edge_refs(i, dbufX, sbufX):
            ib = pl.multiple_of(i * CH, 8)
            return ((dst_h.at[pl.ds(ib, CH)], dbufX),
                    (src_h.at[pl.ds(ib, CH)], sbufX))

        def fire_edges(i, dbufX, sbufX, semX):
            for src, dst in edge_refs(i, dbufX, sbufX):
                pltpu.async_copy(src, dst, semX)

        def wait_edges(i, dbufX, sbufX, semX):
            for src, dst in edge_refs(i, dbufX, sbufX):
                pltpu.make_async_copy(src, dst, semX).wait()

        NCH = E // CH
        for src, dst in edge_refs(0, dbufA, sbufA):
            pltpu.sync_copy(src, dst)
        fire_edges(1, dbufB, sbufB, semB)

        def outer(p, carry):
            iA = 2 * p
            carry = lax.fori_loop(0, CH // 16, make_inner(dbufA, sbufA),
                                  carry)

            @pl.when(iA + 2 < NCH)
            def _():
                fire_edges(iA + 2, dbufA, sbufA, semA)
            wait_edges(iA + 1, dbufB, sbufB, semB)
            carry = lax.fori_loop(0, CH // 16, make_inner(dbufB, sbufB),
                                  carry)

            @pl.when(iA + 3 < NCH)
            def _():
                fire_edges(iA + 3, dbufB, sbufB, semB)

            @pl.when(iA + 2 < NCH)
            def _():
                wait_edges(iA + 2, dbufA, sbufA, semA)
            return carry

        w, base = lax.fori_loop(0, NCH // 2, outer,
                                (jnp.int32(0), jnp.int32(0)))

        wp = ((w + (C - 1)) // C) * C

        def padb(k, carry):
            csrc[pl.ds(w + k * 16, 16)] = jnp.zeros((16,), jnp.int32)
            cdst[pl.ds(w + k * 16, 16)] = jnp.full((16,), R, jnp.int32)
            return carry
        lax.fori_loop(0, (wp - w + 15) // 16, padb, 0)

        def drain(k, carry):
            db = pl.multiple_of(base + k * C, C)
            pltpu.sync_copy(csrc.at[pl.ds(k * C, C)],
                            srcl_h.at[wid, pl.ds(db, C)])
            pltpu.sync_copy(cdst.at[pl.ds(k * C, C)],
                            dstl_h.at[wid, pl.ds(db, C)])
            return carry
        lax.fori_loop(0, wp // C, drain, 0)

        cntv[...] = jnp.full((16,), base + wp, jnp.int32)
        pltpu.sync_copy(cntv, cnt_h.at[wid])
        pltpu.sync_copy(degv, deg_h.at[wid])

    return bucket


@functools.cache
def _acc_kernel(E, npass):
    ACC = RT * FW
    K = FW // 16
    mesh = plsc.VectorSubcoreMesh(core_axis_name="c", subcore_axis_name="s")
    out_type = tuple(
        jax.ShapeDtypeStruct((npass, NPADR * FW), jnp.float32)
        for _ in range(4))
    scratch = [
        pltpu.VMEM((ACC,), jnp.float32),
        pltpu.VMEM((ACC,), jnp.float32),
        [pltpu.VMEM((RT * 16,), jnp.float32) for _ in range(K)],
        [pltpu.VMEM((RT * 16,), jnp.float32) for _ in range(K)],
        pltpu.VMEM((R * FW,), jnp.float32),
        pltpu.VMEM((CG, FW), jnp.float32),
        pltpu.VMEM((CG, FW), jnp.float32),
        pltpu.VMEM((CG,), jnp.int32),
        pltpu.VMEM((CG,), jnp.int32),
        pltpu.VMEM((CG,), jnp.int32),
        pltpu.VMEM((CG,), jnp.int32),
        pltpu.VMEM((CG,), jnp.int32),
        pltpu.VMEM((CG,), jnp.int32),
        pltpu.VMEM((CG,), jnp.int32),
        pltpu.VMEM((16,), jnp.int32),
        pltpu.SemaphoreType.DMA,
        pltpu.SemaphoreType.DMA,
        pltpu.SemaphoreType.DMA,
        pltpu.SemaphoreType.DMA,
    ]

    @functools.partial(pl.kernel, out_type=out_type, mesh=mesh,
                       scratch_types=scratch,
                       compiler_params=pltpu.CompilerParams(
                           use_tc_tiling_on_sc=False,
                           needs_layout_passes=False))
    def acc(table_h, srcl_h, dstl_h, cnt_h, s1_h, s2_h, mn_h, mx_h,
            accS, accQ, mnk, mxk, merge, rowsA, rowsB,
            sbufA, sbufB, dbufA, dbufB, giA, giB, dacc, cntv,
            semLA, semLB, semGA, semGB):
        wid = _wid()
        lo = wid * R
        pltpu.sync_copy(cnt_h.at[wid], cntv)
        cnt = jnp.max(cntv[...])
        nch = cnt // CG

        def list_refs(c_idx, sbufX, dbufX):
            eb = pl.multiple_of(c_idx * CG, CG)
            return ((srcl_h.at[wid, pl.ds(eb, CG)], sbufX),
                    (dstl_h.at[wid, pl.ds(eb, CG)], dbufX))

        def fire_list(c_idx, sbufX, dbufX, semX):
            for src, dst in list_refs(c_idx, sbufX, dbufX):
                pltpu.async_copy(src, dst, semX)

        def wait_list(c_idx, sbufX, dbufX, semX):
            for src, dst in list_refs(c_idx, sbufX, dbufX):
                pltpu.make_async_copy(src, dst, semX).wait()

        for f in range(npass):
            def fire_gather(sbufX, giX, rowsX, semX):
                for k in range(CG // 16):
                    giX[pl.ds(k * 16, 16)] = \
                        sbufX[pl.ds(k * 16, 16)] * npass + f
                pltpu.async_copy(table_h.at[giX], rowsX, semX)

            def wait_gather(giX, rowsX, semX):
                pltpu.make_async_copy(table_h.at[giX], rowsX, semX).wait()

            def do_acc(rowsX):
                def edge(g, carry2):
                    dv = dacc[pl.ds(g * 16, 16)]
                    for l in range(16):
                        dl = dv[l]
                        off64 = dl * FW
                        off16 = dl * 16
                        e = g * 16 + l
                        for k in range(K):
                            r = rowsX[e, pl.ds(k * 16, 16)]
                            plsc.addupdate(
                                accS.at[pl.ds(off64 + k * 16, 16)], r)
                            plsc.addupdate(
                                accQ.at[pl.ds(off64 + k * 16, 16)], r * r)
                            cm = mnk[k][pl.ds(off16, 16)]
                            mnk[k][pl.ds(off16, 16)] = jnp.minimum(cm, r)
                            cx = mxk[k][pl.ds(off16, 16)]
                            mxk[k][pl.ds(off16, 16)] = jnp.maximum(cx, r)
                    return carry2
                lax.fori_loop(0, CG // 16, edge, 0)

            def copy_dst(dbufX):
                for k in range(CG // 16):
                    dacc[pl.ds(k * 16, 16)] = dbufX[pl.ds(k * 16, 16)]

            def zb(i, carry):
                z = jnp.zeros((16,), jnp.float32)
                accS[pl.ds(i * 16, 16)] = z
                accQ[pl.ds(i * 16, 16)] = z
                return carry
            lax.fori_loop(0, ACC // 16, zb, 0)

            def zk(i, carry):
                for k in range(K):
                    mnk[k][pl.ds(i * 16, 16)] = jnp.full((16,), 3e38,
                                                         jnp.float32)
                    mxk[k][pl.ds(i * 16, 16)] = jnp.full((16,), -3e38,
                                                         jnp.float32)
                return carry
            lax.fori_loop(0, RT, zk, 0)

            @pl.when(nch > 0)
            def _():
                for src, dst in list_refs(0, sbufA, dbufA):
                    pltpu.sync_copy(src, dst)
                fire_gather(sbufA, giA, rowsA, semGA)

            @pl.when(nch > 1)
            def _():
                fire_list(1, sbufB, dbufB, semLB)

            def pair(p, carry):
                cA = 2 * p
                cB = cA + 1
                copy_dst(dbufA)

                @pl.when(cB < nch)
                def _():
                    wait_list(cB, sbufB, dbufB, semLB)
                    fire_gather(sbufB, giB, rowsB, semGB)

                @pl.when(cA + 2 < nch)
                def _():
                    fire_list(cA + 2, sbufA, dbufA, semLA)

                wait_gather(giA, rowsA, semGA)
                do_acc(rowsA)

                @pl.when(cB < nch)
                def _():
                    copy_dst(dbufB)

                    @pl.when(cB + 2 < nch)
                    def _():
                        fire_list(cB + 2, sbufB, dbufB, semLB)

                    wait_gather(giB, rowsB, semGB)
                    do_acc(rowsB)

                    @pl.when(cA + 2 < nch)
                    def _():
                        wait_list(cA + 2, sbufA, dbufA, semLA)
                        fire_gather(sbufA, giA, rowsA, semGA)

                return carry
            lax.fori_loop(0, (nch + 1) // 2, pair, 0)

            ob = pl.multiple_of(lo * FW, FW)
            pltpu.sync_copy(accS.at[pl.ds(0, R * FW)],
                            s1_h.at[f, pl.ds(ob, R * FW)])
            pltpu.sync_copy(accQ.at[pl.ds(0, R * FW)],
                            s2_h.at[f, pl.ds(ob, R * FW)])
            for kref, out_h in ((mnk, mn_h), (mxk, mx_h)):
                def mg(i, carry):
                    for k in range(K):
                        merge[pl.ds(i * FW + k * 16, 16)] = \
                            kref[k][pl.ds(i * 16, 16)]
                    return carry
                lax.fori_loop(0, R, mg, 0)
                pltpu.sync_copy(merge, out_h.at[f, pl.ds(ob, R * FW)])

    return acc


def _tc_pre(x, Wd, Ws, b, blk=2000):
    n, fin = x.shape
    fo = Wd.shape[1]

    def body(xr, wdr, wsr, br, ar, btr):
        xb = xr[...]
        ar[...] = jnp.dot(xb, wdr[...],
                          preferred_element_type=jnp.float32) + br[...]
        btr[...] = jnp.dot(xb, wsr[...], preferred_element_type=jnp.float32)

    return pl.pallas_call(
        body,
        grid=(n // blk,),
        in_specs=[
            pl.BlockSpec((blk, fin), lambda i: (i, 0)),
            pl.BlockSpec((fin, fo), lambda i: (0, 0)),
            pl.BlockSpec((fin, fo), lambda i: (0, 0)),
            pl.BlockSpec((1, fo), lambda i: (0, 0)),
        ],
        out_specs=[
            pl.BlockSpec((blk, fo), lambda i: (i, 0)),
            pl.BlockSpec((blk, fo), lambda i: (i, 0)),
        ],
        out_shape=[jax.ShapeDtypeStruct((n, fo), jnp.float32)] * 2,
    )(x, Wd, Ws, b.reshape(1, fo))


def _tc_avglog(degp, n):
    def body(degr, outr):
        d = degr[...]
        col = lax.broadcasted_iota(jnp.int32, (NT, DEGW), 1)
        row = lax.broadcasted_iota(jnp.int32, (NT, DEGW), 0)
        valid = (col < R) & (row * R + col < n)
        outr[0, 0] = jnp.sum(jnp.where(valid, jnp.log(d + 1.0), 0.0)) / n

    return pl.pallas_call(
        body,
        in_specs=[pl.BlockSpec((NT, DEGW), lambda: (0, 0))],
        out_specs=pl.BlockSpec(memory_space=pltpu.SMEM),
        out_shape=jax.ShapeDtypeStruct((1, 1), jnp.float32),
    )(degp)


def _tc_post(xin, A, s1, s2, mn, mx, deg, avgl, postW, postb, linW, linb,
             avg_lin, pre_next=None, blk=1000):
    n, fin = xin.shape
    f = A.shape[1]
    npass = s1.shape[0]
    hid = postW.shape[1]
    ho = linW.shape[1]
    wx = postW[:f]
    w1 = postW[f:5 * f]
    w2 = postW[5 * f:9 * f]
    w3 = postW[9 * f:13 * f]
    w4 = postW[13 * f:17 * f]
    fused = pre_next is not None
    if fused:
        pnW, pnb = pre_next
        fn = pnW.shape[1]

    def body(xr, ar, s1r, s2r, mnr, mxr, degr, avr,
             wxr, w1r, w2r, w3r, w4r, pbr, lwr, lbr, *rest):
        deg_b = degr[...]
        dc = jnp.maximum(deg_b, 1.0)
        has = deg_b > 0.0
        cA = ar[...]
        if npass == 2:
            s1v = jnp.concatenate([s1r[0], s1r[1]], axis=-1)
            s2v = jnp.concatenate([s2r[0], s2r[1]], axis=-1)
            mnv = jnp.concatenate([mnr[0], mnr[1]], axis=-1)
            mxv = jnp.concatenate([mxr[0], mxr[1]], axis=-1)
        else:
            s1v, s2v, mnv, mxv = s1r[0], s2r[0], mnr[0], mxr[0]
        s1d = s1v / dc
        mean = jnp.where(has, cA + s1d, 0.0)
        mnx = jnp.where(has, cA + mnv, 0.0)
        mxx = jnp.where(has, cA + mxv, 0.0)
        var = jnp.maximum(s2v / dc - s1d * s1d, 0.0)
        std = jnp.sqrt(var + 1e-5)
        aggr = jnp.concatenate([mean, mnx, mxx, std], axis=-1)
        al = avr[0, 0]
        ld = jnp.log(dc + 1.0)
        o = jnp.dot(xr[...], wxr[...], preferred_element_type=jnp.float32)
        o += jnp.dot(aggr, w1r[...], preferred_element_type=jnp.float32)
        o += jnp.dot(aggr * (ld / al), w2r[...],
                     preferred_element_type=jnp.float32)
        o += jnp.dot(aggr * (al / ld), w3r[...],
                     preferred_element_type=jnp.float32)
        o += jnp.dot(aggr * (dc / avg_lin), w4r[...],
                     preferred_element_type=jnp.float32)
        o += pbr[...]
        o = jnp.dot(o, lwr[...], preferred_element_type=jnp.float32)
        o += lbr[...]
        if fused:
            wdr, wsr, pnbr, hr, a2r, b2r = rest
            h = jnp.maximum(o, 0.0)
            hr[...] = h
            a2r[...] = jnp.dot(h, wdr[...],
                               preferred_element_type=jnp.float32) + pnbr[...]
            b2r[...] = jnp.dot(h, wsr[...],
                               preferred_element_type=jnp.float32)
        else:
            outr, = rest
            om = o - jnp.max(o, axis=1, keepdims=True)
            outr[...] = om - jnp.log(
                jnp.sum(jnp.exp(om), axis=1, keepdims=True))

    def full(a):
        return pl.BlockSpec(a.shape, lambda i: (0,) * a.ndim)

    in_specs = [
        pl.BlockSpec((blk, fin), lambda i: (i, 0)),
        pl.BlockSpec((blk, f), lambda i: (i, 0)),
    ] + [pl.BlockSpec((npass, blk, FW), lambda i: (0, i, 0))] * 4 + [
        pl.BlockSpec((blk, 1), lambda i: (i, 0)),
        pl.BlockSpec(memory_space=pltpu.SMEM),
        full(wx), full(w1), full(w2), full(w3), full(w4),
        pl.BlockSpec((1, hid), lambda i: (0, 0)),
        full(linW),
        pl.BlockSpec((1, ho), lambda i: (0, 0)),
    ]
    args = [xin, A, s1, s2, mn, mx, deg, avgl,
            wx, w1, w2, w3, w4, postb.reshape(1, hid), linW,
            linb.reshape(1, ho)]
    if fused:
        in_specs += [full(pnW[:hid]), full(pnW[hid:]),
                     pl.BlockSpec((1, fn), lambda i: (0, 0))]
        args += [pnW[:hid], pnW[hid:], pnb.reshape(1, fn)]
        out_specs = [pl.BlockSpec((blk, ho), lambda i: (i, 0)),
                     pl.BlockSpec((blk, fn), lambda i: (i, 0)),
                     pl.BlockSpec((blk, fn), lambda i: (i, 0))]
        out_shape = [jax.ShapeDtypeStruct((n, ho), jnp.float32),
                     jax.ShapeDtypeStruct((n, fn), jnp.float32),
                     jax.ShapeDtypeStruct((n, fn), jnp.float32)]
    else:
        out_specs = [pl.BlockSpec((blk, ho), lambda i: (i, 0))]
        out_shape = [jax.ShapeDtypeStruct((n, ho), jnp.float32)]

    return pl.pallas_call(
        body,
        grid=(n // blk,),
        in_specs=in_specs,
        out_specs=out_specs,
        out_shape=out_shape,
    )(*args)


def kernel(x, edge_index, pre1_W, pre1_b, post1_W, post1_b, lin1_W, lin1_b,
           pre2_W, pre2_b, post2_W, post2_b, lin2_W, lin2_b):
    n, f_in = x.shape
    e = edge_index.shape[1]
    f1 = pre1_W.shape[1]
    avg_lin = float(e) / float(n)

    srcl, dstl, cnth, degp = _bucket_kernel(e)(edge_index[0],
                                               edge_index[1])
    deg = degp[:, :R].reshape(-1)[:n].reshape(n, 1)
    avgl = _tc_avglog(degp, n)

    a1, b1 = _tc_pre(x, pre1_W[:f_in], pre1_W[f_in:], pre1_b)
    np1 = f1 // FW
    table1 = b1.reshape(np1 * n, FW)
    s1, s2, mn, mx = _acc_kernel(e, np1)(table1, srcl, dstl, cnth)

    def rs1(t):
        return t.reshape(np1, NPADR, FW)

    h, a2, b2 = _tc_post(x, a1, rs1(s1), rs1(s2), rs1(mn), rs1(mx), deg,
                         avgl, post1_W, post1_b, lin1_W, lin1_b, avg_lin,
                         pre_next=(pre2_W, pre2_b))

    f2 = pre2_W.shape[1]
    np2 = f2 // FW
    s1b, s2b, mnb, mxb = _acc_kernel(e, np2)(b2, srcl, dstl, cnth)

    def rs2(t):
        return t.reshape(np2, NPADR, FW)

    (out,) = _tc_post(h, a2, rs2(s1b), rs2(s2b), rs2(mnb), rs2(mxb), deg,
                      avgl, post2_W, post2_b, lin2_W, lin2_b, avg_lin)
    return out


# speedup vs baseline: 4.1409x; 1.0459x over previous
import functools

import jax
import jax.numpy as jnp
from jax import lax
from jax.experimental import pallas as pl
from jax.experimental.pallas import tpu as pltpu
from jax.experimental.pallas import tpu_sc as plsc

NT = 32
R = 313
RT = R + 1
NPADR = NT * R
DEGW = 320
CH = 4000
FLUSH = 2048
C = 256
CG = 128
FW = 64


def _wid():
    return lax.axis_index("s") * 2 + lax.axis_index("c")


@functools.cache
def _bucket_kernel(E):
    EPAD = E + C
    mesh = plsc.VectorSubcoreMesh(core_axis_name="c", subcore_axis_name="s")
    out_type = (
        jax.ShapeDtypeStruct((NT, EPAD), jnp.int32),
        jax.ShapeDtypeStruct((NT, EPAD), jnp.int32),
        jax.ShapeDtypeStruct((NT, 16), jnp.int32),
        jax.ShapeDtypeStruct((NT, DEGW), jnp.float32),
    )
    scratch = [
        pltpu.VMEM((CH,), jnp.int32),
        pltpu.VMEM((CH,), jnp.int32),
        pltpu.VMEM((CH,), jnp.int32),
        pltpu.VMEM((CH,), jnp.int32),
        pltpu.VMEM((FLUSH + 2 * C,), jnp.int32),
        pltpu.VMEM((FLUSH + 2 * C,), jnp.int32),
        pltpu.VMEM((DEGW,), jnp.float32),
        pltpu.VMEM((16,), jnp.int32),
        pltpu.SemaphoreType.DMA,
        pltpu.SemaphoreType.DMA,
    ]

    @functools.partial(pl.kernel, out_type=out_type, mesh=mesh,
                       scratch_types=scratch,
                       compiler_params=pltpu.CompilerParams(
                           use_tc_tiling_on_sc=False,
                           needs_layout_passes=False))
    def bucket(src_h, dst_h, srcl_h, dstl_h, cnt_h, deg_h,
               dbufA, sbufA, dbufB, sbufB, csrc, cdst, degv, cntv,
               semA, semB):
        wid = _wid()
        lo = wid * R

        def zb(i, carry):
            degv[pl.ds(i * 16, 16)] = jnp.zeros((16,), jnp.float32)
            return carry
        lax.fori_loop(0, DEGW // 16, zb, 0)

        ones = jnp.ones((16,), jnp.float32)

        def make_inner(dbufX, sbufX):
            def inner(j, carry):
                w, base = carry
                d = dbufX[pl.ds(j * 16, 16)]
                s = sbufX[pl.ds(j * 16, 16)]
                dl = d - lo
                m = (dl >= 0) & (dl < R)
                dlc = jnp.where(m, dl, R)
                plsc.addupdate_scatter(degv, [dlc], ones, mask=m)
                plsc.store_compressed(csrc.at[pl.ds(w, 16)], s, mask=m)
                plsc.store_compressed(cdst.at[pl.ds(w, 16)], dlc, mask=m)
                w = w + plsc.all_reduce_population_count(m)[0]
                fl = w >= FLUSH

                @pl.when(fl)
                def _():
                    fb = pl.multiple_of(base, FLUSH)
                    pltpu.sync_copy(csrc.at[pl.ds(0, FLUSH)],
                                    srcl_h.at[wid, pl.ds(fb, FLUSH)])
                    pltpu.sync_copy(cdst.at[pl.ds(0, FLUSH)],
                                    dstl_h.at[wid, pl.ds(fb, FLUSH)])
                    rs = csrc[pl.ds(FLUSH, 16)]
                    csrc[pl.ds(0, 16)] = rs
                    rd = cdst[pl.ds(FLUSH, 16)]
                    cdst[pl.ds(0, 16)] = rd

                w = jnp.where(fl, w - FLUSH, w)
                base = jnp.where(fl, base + FLUSH, base)
                return (w, base)
            return inner

        def edge_refs(i, dbufX, sbufX):
            ib = pl.multiple_of(i * CH, 8)
            return ((dst_h.at[pl.ds(ib, CH)], dbufX),
                    (src_h.at[pl.ds(ib, CH)], sbufX))

        def fire_edges(i, dbufX, sbufX, semX):
            for src, dst in edge_refs(i, dbufX, sbufX):
                pltpu.async_copy(src, dst, semX)

        def wait_edges(i, dbufX, sbufX, semX):
            for src, dst in edge_refs(i, dbufX, sbufX):
                pltpu.make_async_copy(src, dst, semX).wait()

        NCH = E // CH
        for src, dst in edge_refs(0, dbufA, sbufA):
            pltpu.sync_copy(src, dst)
        fire_edges(1, dbufB, sbufB, semB)

        def outer(p, carry):
            iA = 2 * p
            carry = lax.fori_loop(0, CH // 16, make_inner(dbufA, sbufA),
                                  carry)

            @pl.when(iA + 2 < NCH)
            def _():
                fire_edges(iA + 2, dbufA, sbufA, semA)
            wait_edges(iA + 1, dbufB, sbufB, semB)
            carry = lax.fori_loop(0, CH // 16, make_inner(dbufB, sbufB),
                                  carry)

            @pl.when(iA + 3 < NCH)
            def _():
                fire_edges(iA + 3, dbufB, sbufB, semB)

            @pl.when(iA + 2 < NCH)
            def _():
                wait_edges(iA + 2, dbufA, sbufA, semA)
            return carry

        w, base = lax.fori_loop(0, NCH // 2, outer,
                                (jnp.int32(0), jnp.int32(0)))

        wp = ((w + (C - 1)) // C) * C

        def padb(k, carry):
            csrc[pl.ds(w + k * 16, 16)] = jnp.zeros((16,), jnp.int32)
            cdst[pl.ds(w + k * 16, 16)] = jnp.full((16,), R, jnp.int32)
            return carry
        lax.fori_loop(0, (wp - w + 15) // 16, padb, 0)

        def drain(k, carry):
            db = pl.multiple_of(base + k * C, C)
            pltpu.sync_copy(csrc.at[pl.ds(k * C, C)],
                            srcl_h.at[wid, pl.ds(db, C)])
            pltpu.sync_copy(cdst.at[pl.ds(k * C, C)],
                            dstl_h.at[wid, pl.ds(db, C)])
            return carry
        lax.fori_loop(0, wp // C, drain, 0)

        cntv[...] = jnp.full((16,), base + wp, jnp.int32)
        pltpu.sync_copy(cntv, cnt_h.at[wid])
        pltpu.sync_copy(degv, deg_h.at[wid])

    return bucket


@functools.cache
def _acc_kernel(E, npass):
    K = FW // 16
    mesh = plsc.VectorSubcoreMesh(core_axis_name="c", subcore_axis_name="s")
    out_type = tuple(
        jax.ShapeDtypeStruct((npass, NPADR, FW), jnp.float32)
        for _ in range(4))
    scratch = [
        pltpu.VMEM_SHARED((16 * RT, FW), jnp.float32),
        pltpu.VMEM((RT, FW), jnp.float32),
        [pltpu.VMEM((RT * 16,), jnp.float32) for _ in range(K)],
        [pltpu.VMEM((RT * 16,), jnp.float32) for _ in range(K)],
        pltpu.VMEM((R, FW), jnp.float32),
        pltpu.VMEM((CG, FW), jnp.float32),
        pltpu.VMEM((CG, FW), jnp.float32),
        pltpu.VMEM((CG,), jnp.int32),
        pltpu.VMEM((CG,), jnp.int32),
        pltpu.VMEM((CG,), jnp.int32),
        pltpu.VMEM((CG,), jnp.int32),
        pltpu.VMEM((CG,), jnp.int32),
        pltpu.VMEM((CG,), jnp.int32),
        pltpu.VMEM((CG,), jnp.int32),
        pltpu.VMEM((CG,), jnp.int32),
        pltpu.VMEM((CG,), jnp.int32),
        pltpu.VMEM((16,), jnp.int32),
        pltpu.SemaphoreType.DMA,
        pltpu.SemaphoreType.DMA,
        pltpu.SemaphoreType.DMA,
        pltpu.SemaphoreType.DMA,
        pltpu.SemaphoreType.DMA,
        pltpu.SemaphoreType.DMA,
    ]

    @functools.partial(pl.kernel, out_type=out_type, mesh=mesh,
                       scratch_types=scratch,
                       compiler_params=pltpu.CompilerParams(
                           use_tc_tiling_on_sc=False,
                           needs_layout_passes=False))
    def acc(table_h, srcl_h, dstl_h, cnt_h,
            s1_h, s2_h, mn_h, mx_h,
            spmS, accQ, mnk, mxk, merge, rowsA, rowsB,
            sbufA, sbufB, dbufA, dbufB, giA, giB, sxA, sxB, dacc, cntv,
            semLA, semLB, semGA, semGB, semSA, semSB):
        wid = _wid()
        lo = wid * R
        lbase = lax.axis_index("s") * RT
        pltpu.sync_copy(cnt_h.at[wid], cntv)
        cnt = jnp.max(cntv[...])
        nch = cnt // CG

        def list_refs(c_idx, sbufX, dbufX):
            eb = pl.multiple_of(c_idx * CG, CG)
            return ((srcl_h.at[wid, pl.ds(eb, CG)], sbufX),
                    (dstl_h.at[wid, pl.ds(eb, CG)], dbufX))

        def fire_list(c_idx, sbufX, dbufX, semX):
            for src, dst in list_refs(c_idx, sbufX, dbufX):
                pltpu.async_copy(src, dst, semX)

        def wait_list(c_idx, sbufX, dbufX, semX):
            for src, dst in list_refs(c_idx, sbufX, dbufX):
                pltpu.make_async_copy(src, dst, semX).wait()

        for f in range(npass):
            def fire_gather(sbufX, giX, rowsX, semX):
                for k in range(CG // 16):
                    giX[pl.ds(k * 16, 16)] = \
                        sbufX[pl.ds(k * 16, 16)] * npass + f
                pltpu.async_copy(table_h.at[giX], rowsX, semX)

            def wait_gather(giX, rowsX, semX):
                pltpu.make_async_copy(table_h.at[giX], rowsX, semX).wait()

            def fire_scatter(rowsX, sxX, semX):
                pltpu.async_copy(rowsX, spmS.at[sxX], semX, add=True)

            def wait_scatter(rowsX, sxX, semX):
                pltpu.make_async_copy(rowsX, spmS.at[sxX], semX).wait()

            def do_minmax(rowsX):
                def edge(g, carry2):
                    dv = dacc[pl.ds(g * 16, 16)]
                    for l in range(16):
                        dl = dv[l]
                        off16 = dl * 16
                        e = g * 16 + l
                        for k in range(K):
                            r = rowsX[e, pl.ds(k * 16, 16)]
                            plsc.addupdate(
                                accQ.at[dl, pl.ds(k * 16, 16)], r * r)
                            cm = mnk[k][pl.ds(off16, 16)]
                            mnk[k][pl.ds(off16, 16)] = jnp.minimum(cm, r)
                            cx = mxk[k][pl.ds(off16, 16)]
                            mxk[k][pl.ds(off16, 16)] = jnp.maximum(cx, r)
                    return carry2
                lax.fori_loop(0, CG // 16, edge, 0)

            def copy_dst(dbufX, sxX):
                for k in range(CG // 16):
                    dlv = dbufX[pl.ds(k * 16, 16)]
                    dacc[pl.ds(k * 16, 16)] = dlv
                    sxX[pl.ds(k * 16, 16)] = dlv + lbase

            def zm(i, carry):
                for k in range(K):
                    z = jnp.zeros((16,), jnp.float32)
                    merge[i, pl.ds(k * 16, 16)] = z
                    accQ[i, pl.ds(k * 16, 16)] = z
                return carry
            lax.fori_loop(0, R, zm, 0)
            for k in range(K):
                accQ[R, pl.ds(k * 16, 16)] = jnp.zeros((16,), jnp.float32)
            pltpu.sync_copy(merge, spmS.at[pl.ds(lbase, R)])
            pltpu.sync_copy(merge.at[pl.ds(0, 1)],
                            spmS.at[pl.ds(lbase + R, 1)])

            def zk(i, carry):
                for k in range(K):
                    mnk[k][pl.ds(i * 16, 16)] = jnp.full((16,), 3e38,
                                                         jnp.float32)
                    mxk[k][pl.ds(i * 16, 16)] = jnp.full((16,), -3e38,
                                                         jnp.float32)
                return carry
            lax.fori_loop(0, RT, zk, 0)

            @pl.when(nch > 0)
            def _():
                for src, dst in list_refs(0, sbufA, dbufA):
                    pltpu.sync_copy(src, dst)
                fire_gather(sbufA, giA, rowsA, semGA)

            @pl.when(nch > 1)
            def _():
                fire_list(1, sbufB, dbufB, semLB)

            def pair(p, carry):
                cA = 2 * p
                cB = cA + 1
                copy_dst(dbufA, sxA)

                @pl.when(cB < nch)
                def _():
                    wait_list(cB, sbufB, dbufB, semLB)
                    fire_gather(sbufB, giB, rowsB, semGB)

                @pl.when(cA + 2 < nch)
                def _():
                    fire_list(cA + 2, sbufA, dbufA, semLA)

                wait_gather(giA, rowsA, semGA)
                fire_scatter(rowsA, sxA, semSA)
                do_minmax(rowsA)
                wait_scatter(rowsA, sxA, semSA)

                @pl.when(cB < nch)
                def _():
                    copy_dst(dbufB, sxB)

                    @pl.when(cB + 2 < nch)
                    def _():
                        fire_list(cB + 2, sbufB, dbufB, semLB)

                    wait_gather(giB, rowsB, semGB)
                    fire_scatter(rowsB, sxB, semSB)
                    do_minmax(rowsB)
                    wait_scatter(rowsB, sxB, semSB)

                    @pl.when(cA + 2 < nch)
                    def _():
                        wait_list(cA + 2, sbufA, dbufA, semLA)
                        fire_gather(sbufA, giA, rowsA, semGA)

                return carry
            lax.fori_loop(0, (nch + 1) // 2, pair, 0)

            pltpu.sync_copy(spmS.at[pl.ds(lbase, R)],
                            s1_h.at[f, pl.ds(lo, R)])
            pltpu.sync_copy(accQ.at[pl.ds(0, R)],
                            s2_h.at[f, pl.ds(lo, R)])
            for kref, out_h in ((mnk, mn_h), (mxk, mx_h)):
                def mg(i, carry):
                    for k in range(K):
                        merge[i, pl.ds(k * 16, 16)] = \
                            kref[k][pl.ds(i * 16, 16)]
                    return carry
                lax.fori_loop(0, R, mg, 0)
                pltpu.sync_copy(merge, out_h.at[f, pl.ds(lo, R)])

    return acc


def _tc_pre(x, Wd, Ws, b, blk=2000):
    n, fin = x.shape
    fo = Wd.shape[1]

    def body(xr, wdr, wsr, br, ar, btr):
        xb = xr[...]
        ar[...] = jnp.dot(xb, wdr[...],
                          preferred_element_type=jnp.float32) + br[...]
        btr[...] = jnp.dot(xb, wsr[...], preferred_element_type=jnp.float32)

    return pl.pallas_call(
        body,
        grid=(n // blk,),
        in_specs=[
            pl.BlockSpec((blk, fin), lambda i: (i, 0)),
            pl.BlockSpec((fin, fo), lambda i: (0, 0)),
            pl.BlockSpec((fin, fo), lambda i: (0, 0)),
            pl.BlockSpec((1, fo), lambda i: (0, 0)),
        ],
        out_specs=[
            pl.BlockSpec((blk, fo), lambda i: (i, 0)),
            pl.BlockSpec((blk, fo), lambda i: (i, 0)),
        ],
        out_shape=[jax.ShapeDtypeStruct((n, fo), jnp.float32)] * 2,
    )(x, Wd, Ws, b.reshape(1, fo))


def _tc_avglog(degp, n):
    def body(degr, outr):
        d = degr[...]
        col = lax.broadcasted_iota(jnp.int32, (NT, DEGW), 1)
        row = lax.broadcasted_iota(jnp.int32, (NT, DEGW), 0)
        valid = (col < R) & (row * R + col < n)
        outr[0, 0] = jnp.sum(jnp.where(valid, jnp.log(d + 1.0), 0.0)) / n

    return pl.pallas_call(
        body,
        in_specs=[pl.BlockSpec((NT, DEGW), lambda: (0, 0))],
        out_specs=pl.BlockSpec(memory_space=pltpu.SMEM),
        out_shape=jax.ShapeDtypeStruct((1, 1), jnp.float32),
    )(degp)


def _tc_post(xin, A, s1, s2, mn, mx, deg, avgl, postW, postb, linW, linb,
             avg_lin, pre_next=None, blk=1000):
    n, fin = xin.shape
    f = A.shape[1]
    npass = s1.shape[0]
    hid = postW.shape[1]
    ho = linW.shape[1]
    wx = postW[:f]
    w1 = postW[f:5 * f]
    w2 = postW[5 * f:9 * f]
    w3 = postW[9 * f:13 * f]
    w4 = postW[13 * f:17 * f]
    fused = pre_next is not None
    if fused:
        pnW, pnb = pre_next
        fn = pnW.shape[1]

    def body(xr, ar, s1r, s2r, mnr, mxr, degr, avr,
             wxr, w1r, w2r, w3r, w4r, pbr, lwr, lbr, *rest):
        deg_b = degr[...]
        dc = jnp.maximum(deg_b, 1.0)
        has = deg_b > 0.0
        cA = ar[...]
        if npass == 2:
            s1v = jnp.concatenate([s1r[0], s1r[1]], axis=-1)
            s2v = jnp.concatenate([s2r[0], s2r[1]], axis=-1)
            mnv = jnp.concatenate([mnr[0], mnr[1]], axis=-1)
            mxv = jnp.concatenate([mxr[0], mxr[1]], axis=-1)
        else:
            s1v, s2v, mnv, mxv = s1r[0], s2r[0], mnr[0], mxr[0]
        s1d = s1v / dc
        mean = jnp.where(has, cA + s1d, 0.0)
        mnx = jnp.where(has, cA + mnv, 0.0)
        mxx = jnp.where(has, cA + mxv, 0.0)
        var = jnp.maximum(s2v / dc - s1d * s1d, 0.0)
        std = jnp.sqrt(var + 1e-5)
        aggr = jnp.concatenate([mean, mnx, mxx, std], axis=-1)
        al = avr[0, 0]
        ld = jnp.log(dc + 1.0)
        o = jnp.dot(xr[...], wxr[...], preferred_element_type=jnp.float32)
        o += jnp.dot(aggr, w1r[...], preferred_element_type=jnp.float32)
        o += jnp.dot(aggr * (ld / al), w2r[...],
                     preferred_element_type=jnp.float32)
        o += jnp.dot(aggr * (al / ld), w3r[...],
                     preferred_element_type=jnp.float32)
        o += jnp.dot(aggr * (dc / avg_lin), w4r[...],
                     preferred_element_type=jnp.float32)
        o += pbr[...]
        o = jnp.dot(o, lwr[...], preferred_element_type=jnp.float32)
        o += lbr[...]
        if fused:
            wdr, wsr, pnbr, hr, a2r, b2r = rest
            h = jnp.maximum(o, 0.0)
            hr[...] = h
            a2r[...] = jnp.dot(h, wdr[...],
                               preferred_element_type=jnp.float32) + pnbr[...]
            b2r[...] = jnp.dot(h, wsr[...],
                               preferred_element_type=jnp.float32)
        else:
            outr, = rest
            om = o - jnp.max(o, axis=1, keepdims=True)
            outr[...] = om - jnp.log(
                jnp.sum(jnp.exp(om), axis=1, keepdims=True))

    def full(a):
        return pl.BlockSpec(a.shape, lambda i: (0,) * a.ndim)

    in_specs = [
        pl.BlockSpec((blk, fin), lambda i: (i, 0)),
        pl.BlockSpec((blk, f), lambda i: (i, 0)),
    ] + [pl.BlockSpec((npass, blk, FW), lambda i: (0, i, 0))] * 4 + [
        pl.BlockSpec((blk, 1), lambda i: (i, 0)),
        pl.BlockSpec(memory_space=pltpu.SMEM),
        full(wx), full(w1), full(w2), full(w3), full(w4),
        pl.BlockSpec((1, hid), lambda i: (0, 0)),
        full(linW),
        pl.BlockSpec((1, ho), lambda i: (0, 0)),
    ]
    args = [xin, A, s1, s2, mn, mx, deg, avgl,
            wx, w1, w2, w3, w4, postb.reshape(1, hid), linW,
            linb.reshape(1, ho)]
    if fused:
        in_specs += [full(pnW[:hid]), full(pnW[hid:]),
                     pl.BlockSpec((1, fn), lambda i: (0, 0))]
        args += [pnW[:hid], pnW[hid:], pnb.reshape(1, fn)]
        out_specs = [pl.BlockSpec((blk, ho), lambda i: (i, 0)),
                     pl.BlockSpec((blk, fn), lambda i: (i, 0)),
                     pl.BlockSpec((blk, fn), lambda i: (i, 0))]
        out_shape = [jax.ShapeDtypeStruct((n, ho), jnp.float32),
                     jax.ShapeDtypeStruct((n, fn), jnp.float32),
                     jax.ShapeDtypeStruct((n, fn), jnp.float32)]
    else:
        out_specs = [pl.BlockSpec((blk, ho), lambda i: (i, 0))]
        out_shape = [jax.ShapeDtypeStruct((n, ho), jnp.float32)]

    return pl.pallas_call(
        body,
        grid=(n // blk,),
        in_specs=in_specs,
        out_specs=out_specs,
        out_shape=out_shape,
    )(*args)


def kernel(x, edge_index, pre1_W, pre1_b, post1_W, post1_b, lin1_W, lin1_b,
           pre2_W, pre2_b, post2_W, post2_b, lin2_W, lin2_b):
    n, f_in = x.shape
    e = edge_index.shape[1]
    f1 = pre1_W.shape[1]
    avg_lin = float(e) / float(n)

    srcl, dstl, cnth, degp = _bucket_kernel(e)(edge_index[0],
                                               edge_index[1])
    deg = degp[:, :R].reshape(-1)[:n].reshape(n, 1)
    avgl = _tc_avglog(degp, n)

    a1, b1 = _tc_pre(x, pre1_W[:f_in], pre1_W[f_in:], pre1_b)
    np1 = f1 // FW
    s1, s2, mn, mx = _acc_kernel(e, np1)(
        b1.reshape(np1 * n, FW), srcl, dstl, cnth)

    h, a2, b2 = _tc_post(x, a1, s1, s2, mn, mx, deg,
                         avgl, post1_W, post1_b, lin1_W, lin1_b,
                         avg_lin, pre_next=(pre2_W, pre2_b))

    f2 = pre2_W.shape[1]
    np2 = f2 // FW
    s1b, s2b, mnb, mxb = _acc_kernel(e, np2)(b2, srcl, dstl, cnth)

    (out,) = _tc_post(h, a2, s1b, s2b, mnb, mxb, deg,
                      avgl, post2_W, post2_b, lin2_W, lin2_b, avg_lin)
    return out


# speedup vs baseline: 4.5091x; 1.0889x over previous
import functools

import jax
import jax.numpy as jnp
from jax import lax
from jax.experimental import pallas as pl
from jax.experimental.pallas import tpu as pltpu
from jax.experimental.pallas import tpu_sc as plsc

NT = 32
R = 313
RT = R + 1
NPADR = NT * R
DEGW = 320
CH = 4000
FLUSH = 2048
C = 256
CG = 128
FW = 64


def _wid():
    return lax.axis_index("s") * 2 + lax.axis_index("c")


@functools.cache
def _bucket_kernel(E):
    EPAD = E + C
    mesh = plsc.VectorSubcoreMesh(core_axis_name="c", subcore_axis_name="s")
    out_type = (
        jax.ShapeDtypeStruct((NT, EPAD), jnp.int32),
        jax.ShapeDtypeStruct((NT, EPAD), jnp.int32),
        jax.ShapeDtypeStruct((NT, 16), jnp.int32),
        jax.ShapeDtypeStruct((NT, DEGW), jnp.float32),
    )
    scratch = [
        pltpu.VMEM((CH,), jnp.int32),
        pltpu.VMEM((CH,), jnp.int32),
        pltpu.VMEM((CH,), jnp.int32),
        pltpu.VMEM((CH,), jnp.int32),
        pltpu.VMEM((FLUSH + 2 * C,), jnp.int32),
        pltpu.VMEM((FLUSH + 2 * C,), jnp.int32),
        pltpu.VMEM((DEGW,), jnp.float32),
        pltpu.VMEM((16,), jnp.int32),
        pltpu.SemaphoreType.DMA,
        pltpu.SemaphoreType.DMA,
    ]

    @functools.partial(pl.kernel, out_type=out_type, mesh=mesh,
                       scratch_types=scratch,
                       compiler_params=pltpu.CompilerParams(
                           use_tc_tiling_on_sc=False,
                           needs_layout_passes=False))
    def bucket(src_h, dst_h, srcl_h, dstl_h, cnt_h, deg_h,
               dbufA, sbufA, dbufB, sbufB, csrc, cdst, degv, cntv,
               semA, semB):
        wid = _wid()
        lo = wid * R

        def zb(i, carry):
            degv[pl.ds(i * 16, 16)] = jnp.zeros((16,), jnp.float32)
            return carry
        lax.fori_loop(0, DEGW // 16, zb, 0)

        ones = jnp.ones((16,), jnp.float32)

        GV = 5

        def make_inner(dbufX, sbufX):
            def inner(j, carry):
                wv, base = carry
                for u in range(GV):
                    d = dbufX[pl.ds((j * GV + u) * 16, 16)]
                    s = sbufX[pl.ds((j * GV + u) * 16, 16)]
                    dl = d - lo
                    m = (dl >= 0) & (dl < R)
                    dlc = jnp.where(m, dl, R)
                    plsc.addupdate_scatter(degv, [dlc], ones, mask=m)
                    pos = wv + plsc.cumsum(jnp.where(m, 1, 0)) - 1
                    plsc.store_scatter(csrc, [pos], s, mask=m)
                    plsc.store_scatter(cdst, [pos], dlc, mask=m)
                    wv = wv + plsc.all_reduce_population_count(m)
                w = wv[0]
                fl = w >= FLUSH

                @pl.when(fl)
                def _():
                    fb = pl.multiple_of(base, FLUSH)
                    pltpu.sync_copy(csrc.at[pl.ds(0, FLUSH)],
                                    srcl_h.at[wid, pl.ds(fb, FLUSH)])
                    pltpu.sync_copy(cdst.at[pl.ds(0, FLUSH)],
                                    dstl_h.at[wid, pl.ds(fb, FLUSH)])
                    for u in range(GV):
                        rs = csrc[pl.ds(FLUSH + u * 16, 16)]
                        csrc[pl.ds(u * 16, 16)] = rs
                        rd = cdst[pl.ds(FLUSH + u * 16, 16)]
                        cdst[pl.ds(u * 16, 16)] = rd

                wv = jnp.where(fl, wv - FLUSH, wv)
                base = jnp.where(fl, base + FLUSH, base)
                return (wv, base)
            return inner

        def edge_refs(i, dbufX, sbufX):
            ib = pl.multiple_of(i * CH, 8)
            return ((dst_h.at[pl.ds(ib, CH)], dbufX),
                    (src_h.at[pl.ds(ib, CH)], sbufX))

        def fire_edges(i, dbufX, sbufX, semX):
            for src, dst in edge_refs(i, dbufX, sbufX):
                pltpu.async_copy(src, dst, semX)

        def wait_edges(i, dbufX, sbufX, semX):
            for src, dst in edge_refs(i, dbufX, sbufX):
                pltpu.make_async_copy(src, dst, semX).wait()

        NCH = E // CH
        for src, dst in edge_refs(0, dbufA, sbufA):
            pltpu.sync_copy(src, dst)
        fire_edges(1, dbufB, sbufB, semB)

        NGR = CH // (16 * GV)

        def outer(p, carry):
            iA = 2 * p
            carry = lax.fori_loop(0, NGR, make_inner(dbufA, sbufA), carry)

            @pl.when(iA + 2 < NCH)
            def _():
                fire_edges(iA + 2, dbufA, sbufA, semA)
            wait_edges(iA + 1, dbufB, sbufB, semB)
            carry = lax.fori_loop(0, NGR, make_inner(dbufB, sbufB), carry)

            @pl.when(iA + 3 < NCH)
            def _():
                fire_edges(iA + 3, dbufB, sbufB, semB)

            @pl.when(iA + 2 < NCH)
            def _():
                wait_edges(iA + 2, dbufA, sbufA, semA)
            return carry

        wv, base = lax.fori_loop(0, NCH // 2, outer,
                                 (jnp.zeros((16,), jnp.int32),
                                  jnp.int32(0)))
        w = wv[0]

        wp = ((w + (C - 1)) // C) * C

        def padb(k, carry):
            csrc[pl.ds(w + k * 16, 16)] = jnp.zeros((16,), jnp.int32)
            cdst[pl.ds(w + k * 16, 16)] = jnp.full((16,), R, jnp.int32)
            return carry
        lax.fori_loop(0, (wp - w + 15) // 16, padb, 0)

        def drain(k, carry):
            db = pl.multiple_of(base + k * C, C)
            pltpu.sync_copy(csrc.at[pl.ds(k * C, C)],
                            srcl_h.at[wid, pl.ds(db, C)])
            pltpu.sync_copy(cdst.at[pl.ds(k * C, C)],
                            dstl_h.at[wid, pl.ds(db, C)])
            return carry
        lax.fori_loop(0, wp // C, drain, 0)

        cntv[...] = jnp.full((16,), base + wp, jnp.int32)
        pltpu.sync_copy(cntv, cnt_h.at[wid])
        pltpu.sync_copy(degv, deg_h.at[wid])

    return bucket


@functools.cache
def _acc_kernel(E, npass):
    K = FW // 16
    mesh = plsc.VectorSubcoreMesh(core_axis_name="c", subcore_axis_name="s")
    out_type = tuple(
        jax.ShapeDtypeStruct((npass, NPADR, FW), jnp.float32)
        for _ in range(4))
    scratch = [
        pltpu.VMEM_SHARED((16 * RT, FW), jnp.float32),
        pltpu.VMEM((RT, FW), jnp.float32),
        [pltpu.VMEM((RT * 16,), jnp.float32) for _ in range(K)],
        [pltpu.VMEM((RT * 16,), jnp.float32) for _ in range(K)],
        pltpu.VMEM((R, FW), jnp.float32),
        pltpu.VMEM((CG, FW), jnp.float32),
        pltpu.VMEM((CG, FW), jnp.float32),
        pltpu.VMEM((CG,), jnp.int32),
        pltpu.VMEM((CG,), jnp.int32),
        pltpu.VMEM((CG,), jnp.int32),
        pltpu.VMEM((CG,), jnp.int32),
        pltpu.VMEM((CG,), jnp.int32),
        pltpu.VMEM((CG,), jnp.int32),
        pltpu.VMEM((CG,), jnp.int32),
        pltpu.VMEM((CG,), jnp.int32),
        pltpu.VMEM((CG,), jnp.int32),
        pltpu.VMEM((16,), jnp.int32),
        pltpu.SemaphoreType.DMA,
        pltpu.SemaphoreType.DMA,
        pltpu.SemaphoreType.DMA,
        pltpu.SemaphoreType.DMA,
        pltpu.SemaphoreType.DMA,
        pltpu.SemaphoreType.DMA,
    ]

    @functools.partial(pl.kernel, out_type=out_type, mesh=mesh,
                       scratch_types=scratch,
                       compiler_params=pltpu.CompilerParams(
                           use_tc_tiling_on_sc=False,
                           needs_layout_passes=False))
    def acc(table_h, srcl_h, dstl_h, cnt_h,
            s1_h, s2_h, mn_h, mx_h,
            spmS, accQ, mnk, mxk, merge, rowsA, rowsB,
            sbufA, sbufB, dbufA, dbufB, giA, giB, sxA, sxB, dacc, cntv,
            semLA, semLB, semGA, semGB, semSA, semSB):
        wid = _wid()
        lo = wid * R
        lbase = lax.axis_index("s") * RT
        pltpu.sync_copy(cnt_h.at[wid], cntv)
        cnt = jnp.max(cntv[...])
        nch = cnt // CG

        def list_refs(c_idx, sbufX, dbufX):
            eb = pl.multiple_of(c_idx * CG, CG)
            return ((srcl_h.at[wid, pl.ds(eb, CG)], sbufX),
                    (dstl_h.at[wid, pl.ds(eb, CG)], dbufX))

        def fire_list(c_idx, sbufX, dbufX, semX):
            for src, dst in list_refs(c_idx, sbufX, dbufX):
                pltpu.async_copy(src, dst, semX)

        def wait_list(c_idx, sbufX, dbufX, semX):
            for src, dst in list_refs(c_idx, sbufX, dbufX):
                pltpu.make_async_copy(src, dst, semX).wait()

        for f in range(npass):
            def fire_gather(sbufX, giX, rowsX, semX):
                for k in range(CG // 16):
                    giX[pl.ds(k * 16, 16)] = \
                        sbufX[pl.ds(k * 16, 16)] * npass + f
                pltpu.async_copy(table_h.at[giX], rowsX, semX)

            def wait_gather(giX, rowsX, semX):
                pltpu.make_async_copy(table_h.at[giX], rowsX, semX).wait()

            def fire_scatter(rowsX, sxX, semX):
                pltpu.async_copy(rowsX, spmS.at[sxX], semX, add=True)

            def wait_scatter(rowsX, sxX, semX):
                pltpu.make_async_copy(rowsX, spmS.at[sxX], semX).wait()

            def do_minmax(rowsX):
                def edge(g, carry2):
                    dv = dacc[pl.ds(g * 16, 16)]
                    for l in range(16):
                        dl = dv[l]
                        off16 = dl * 16
                        e = g * 16 + l
                        for k in range(K):
                            r = rowsX[e, pl.ds(k * 16, 16)]
                            plsc.addupdate(
                                accQ.at[dl, pl.ds(k * 16, 16)], r * r)
                            cm = mnk[k][pl.ds(off16, 16)]
                            mnk[k][pl.ds(off16, 16)] = jnp.minimum(cm, r)
                            cx = mxk[k][pl.ds(off16, 16)]
                            mxk[k][pl.ds(off16, 16)] = jnp.maximum(cx, r)
                    return carry2
                lax.fori_loop(0, CG // 16, edge, 0)

            def copy_dst(dbufX, sxX):
                for k in range(CG // 16):
                    dlv = dbufX[pl.ds(k * 16, 16)]
                    dacc[pl.ds(k * 16, 16)] = dlv
                    sxX[pl.ds(k * 16, 16)] = dlv + lbase

            def zm(i, carry):
                for k in range(K):
                    z = jnp.zeros((16,), jnp.float32)
                    merge[i, pl.ds(k * 16, 16)] = z
                    accQ[i, pl.ds(k * 16, 16)] = z
                return carry
            lax.fori_loop(0, R, zm, 0)
            for k in range(K):
                accQ[R, pl.ds(k * 16, 16)] = jnp.zeros((16,), jnp.float32)
            pltpu.sync_copy(merge, spmS.at[pl.ds(lbase, R)])
            pltpu.sync_copy(merge.at[pl.ds(0, 1)],
                            spmS.at[pl.ds(lbase + R, 1)])

            def zk(i, carry):
                for k in range(K):
                    mnk[k][pl.ds(i * 16, 16)] = jnp.full((16,), 3e38,
                                                         jnp.float32)
                    mxk[k][pl.ds(i * 16, 16)] = jnp.full((16,), -3e38,
                                                         jnp.float32)
                return carry
            lax.fori_loop(0, RT, zk, 0)

            @pl.when(nch > 0)
            def _():
                for src, dst in list_refs(0, sbufA, dbufA):
                    pltpu.sync_copy(src, dst)
                fire_gather(sbufA, giA, rowsA, semGA)

            @pl.when(nch > 1)
            def _():
                fire_list(1, sbufB, dbufB, semLB)

            def pair(p, carry):
                cA = 2 * p
                cB = cA + 1
                copy_dst(dbufA, sxA)

                @pl.when(cB < nch)
                def _():
                    wait_list(cB, sbufB, dbufB, semLB)
                    fire_gather(sbufB, giB, rowsB, semGB)

                @pl.when(cA + 2 < nch)
                def _():
                    fire_list(cA + 2, sbufA, dbufA, semLA)

                wait_gather(giA, rowsA, semGA)
                fire_scatter(rowsA, sxA, semSA)
                do_minmax(rowsA)
                wait_scatter(rowsA, sxA, semSA)

                @pl.when(cB < nch)
                def _():
                    copy_dst(dbufB, sxB)

                    @pl.when(cB + 2 < nch)
                    def _():
                        fire_list(cB + 2, sbufB, dbufB, semLB)

                    wait_gather(giB, rowsB, semGB)
                    fire_scatter(rowsB, sxB, semSB)
                    do_minmax(rowsB)
                    wait_scatter(rowsB, sxB, semSB)

                    @pl.when(cA + 2 < nch)
                    def _():
                        wait_list(cA + 2, sbufA, dbufA, semLA)
                        fire_gather(sbufA, giA, rowsA, semGA)

                return carry
            lax.fori_loop(0, (nch + 1) // 2, pair, 0)

            pltpu.sync_copy(spmS.at[pl.ds(lbase, R)],
                            s1_h.at[f, pl.ds(lo, R)])
            pltpu.sync_copy(accQ.at[pl.ds(0, R)],
                            s2_h.at[f, pl.ds(lo, R)])
            for kref, out_h in ((mnk, mn_h), (mxk, mx_h)):
                def mg(i, carry):
                    for k in range(K):
                        merge[i, pl.ds(k * 16, 16)] = \
                            kref[k][pl.ds(i * 16, 16)]
                    return carry
                lax.fori_loop(0, R, mg, 0)
                pltpu.sync_copy(merge, out_h.at[f, pl.ds(lo, R)])

    return acc


def _tc_pre(x, Wd, Ws, b, blk=2000):
    n, fin = x.shape
    fo = Wd.shape[1]

    def body(xr, wdr, wsr, br, ar, btr):
        xb = xr[...]
        ar[...] = jnp.dot(xb, wdr[...],
                          preferred_element_type=jnp.float32) + br[...]
        btr[...] = jnp.dot(xb, wsr[...], preferred_element_type=jnp.float32)

    return pl.pallas_call(
        body,
        grid=(n // blk,),
        in_specs=[
            pl.BlockSpec((blk, fin), lambda i: (i, 0)),
            pl.BlockSpec((fin, fo), lambda i: (0, 0)),
            pl.BlockSpec((fin, fo), lambda i: (0, 0)),
            pl.BlockSpec((1, fo), lambda i: (0, 0)),
        ],
        out_specs=[
            pl.BlockSpec((blk, fo), lambda i: (i, 0)),
            pl.BlockSpec((blk, fo), lambda i: (i, 0)),
        ],
        out_shape=[jax.ShapeDtypeStruct((n, fo), jnp.float32)] * 2,
    )(x, Wd, Ws, b.reshape(1, fo))


def _tc_avglog(degp, n):
    def body(degr, outr):
        d = degr[...]
        col = lax.broadcasted_iota(jnp.int32, (NT, DEGW), 1)
        row = lax.broadcasted_iota(jnp.int32, (NT, DEGW), 0)
        valid = (col < R) & (row * R + col < n)
        outr[0, 0] = jnp.sum(jnp.where(valid, jnp.log(d + 1.0), 0.0)) / n

    return pl.pallas_call(
        body,
        in_specs=[pl.BlockSpec((NT, DEGW), lambda: (0, 0))],
        out_specs=pl.BlockSpec(memory_space=pltpu.SMEM),
        out_shape=jax.ShapeDtypeStruct((1, 1), jnp.float32),
    )(degp)


def _tc_post(xin, A, s1, s2, mn, mx, deg, avgl, postW, postb, linW, linb,
             avg_lin, pre_next=None, blk=1000):
    n, fin = xin.shape
    f = A.shape[1]
    npass = s1.shape[0]
    hid = postW.shape[1]
    ho = linW.shape[1]
    wx = postW[:f]
    w1 = postW[f:5 * f]
    w2 = postW[5 * f:9 * f]
    w3 = postW[9 * f:13 * f]
    w4 = postW[13 * f:17 * f]
    fused = pre_next is not None
    if fused:
        pnW, pnb = pre_next
        fn = pnW.shape[1]

    def body(xr, ar, s1r, s2r, mnr, mxr, degr, avr,
             wxr, w1r, w2r, w3r, w4r, pbr, lwr, lbr, *rest):
        deg_b = degr[...]
        dc = jnp.maximum(deg_b, 1.0)
        has = deg_b > 0.0
        cA = ar[...]
        if npass == 2:
            s1v = jnp.concatenate([s1r[0], s1r[1]], axis=-1)
            s2v = jnp.concatenate([s2r[0], s2r[1]], axis=-1)
            mnv = jnp.concatenate([mnr[0], mnr[1]], axis=-1)
            mxv = jnp.concatenate([mxr[0], mxr[1]], axis=-1)
        else:
            s1v, s2v, mnv, mxv = s1r[0], s2r[0], mnr[0], mxr[0]
        s1d = s1v / dc
        mean = jnp.where(has, cA + s1d, 0.0)
        mnx = jnp.where(has, cA + mnv, 0.0)
        mxx = jnp.where(has, cA + mxv, 0.0)
        var = jnp.maximum(s2v / dc - s1d * s1d, 0.0)
        std = jnp.sqrt(var + 1e-5)
        aggr = jnp.concatenate([mean, mnx, mxx, std], axis=-1)
        al = avr[0, 0]
        ld = jnp.log(dc + 1.0)
        o = jnp.dot(xr[...], wxr[...], preferred_element_type=jnp.float32)
        o += jnp.dot(aggr, w1r[...], preferred_element_type=jnp.float32)
        o += jnp.dot(aggr * (ld / al), w2r[...],
                     preferred_element_type=jnp.float32)
        o += jnp.dot(aggr * (al / ld), w3r[...],
                     preferred_element_type=jnp.float32)
        o += jnp.dot(aggr * (dc / avg_lin), w4r[...],
                     preferred_element_type=jnp.float32)
        o += pbr[...]
        o = jnp.dot(o, lwr[...], preferred_element_type=jnp.float32)
        o += lbr[...]
        if fused:
            wdr, wsr, pnbr, hr, a2r, b2r = rest
            h = jnp.maximum(o, 0.0)
            hr[...] = h
            a2r[...] = jnp.dot(h, wdr[...],
                               preferred_element_type=jnp.float32) + pnbr[...]
            b2r[...] = jnp.dot(h, wsr[...],
                               preferred_element_type=jnp.float32)
        else:
            outr, = rest
            om = o - jnp.max(o, axis=1, keepdims=True)
            outr[...] = om - jnp.log(
                jnp.sum(jnp.exp(om), axis=1, keepdims=True))

    def full(a):
        return pl.BlockSpec(a.shape, lambda i: (0,) * a.ndim)

    in_specs = [
        pl.BlockSpec((blk, fin), lambda i: (i, 0)),
        pl.BlockSpec((blk, f), lambda i: (i, 0)),
    ] + [pl.BlockSpec((npass, blk, FW), lambda i: (0, i, 0))] * 4 + [
        pl.BlockSpec((blk, 1), lambda i: (i, 0)),
        pl.BlockSpec(memory_space=pltpu.SMEM),
        full(wx), full(w1), full(w2), full(w3), full(w4),
        pl.BlockSpec((1, hid), lambda i: (0, 0)),
        full(linW),
        pl.BlockSpec((1, ho), lambda i: (0, 0)),
    ]
    args = [xin, A, s1, s2, mn, mx, deg, avgl,
            wx, w1, w2, w3, w4, postb.reshape(1, hid), linW,
            linb.reshape(1, ho)]
    if fused:
        in_specs += [full(pnW[:hid]), full(pnW[hid:]),
                     pl.BlockSpec((1, fn), lambda i: (0, 0))]
        args += [pnW[:hid], pnW[hid:], pnb.reshape(1, fn)]
        out_specs = [pl.BlockSpec((blk, ho), lambda i: (i, 0)),
                     pl.BlockSpec((blk, fn), lambda i: (i, 0)),
                     pl.BlockSpec((blk, fn), lambda i: (i, 0))]
        out_shape = [jax.ShapeDtypeStruct((n, ho), jnp.float32),
                     jax.ShapeDtypeStruct((n, fn), jnp.float32),
                     jax.ShapeDtypeStruct((n, fn), jnp.float32)]
    else:
        out_specs = [pl.BlockSpec((blk, ho), lambda i: (i, 0))]
        out_shape = [jax.ShapeDtypeStruct((n, ho), jnp.float32)]

    return pl.pallas_call(
        body,
        grid=(n // blk,),
        in_specs=in_specs,
        out_specs=out_specs,
        out_shape=out_shape,
    )(*args)


def kernel(x, edge_index, pre1_W, pre1_b, post1_W, post1_b, lin1_W, lin1_b,
           pre2_W, pre2_b, post2_W, post2_b, lin2_W, lin2_b):
    n, f_in = x.shape
    e = edge_index.shape[1]
    f1 = pre1_W.shape[1]
    avg_lin = float(e) / float(n)

    srcl, dstl, cnth, degp = _bucket_kernel(e)(edge_index[0],
                                               edge_index[1])
    deg = degp[:, :R].reshape(-1)[:n].reshape(n, 1)
    avgl = _tc_avglog(degp, n)

    a1, b1 = _tc_pre(x, pre1_W[:f_in], pre1_W[f_in:], pre1_b)
    np1 = f1 // FW
    s1, s2, mn, mx = _acc_kernel(e, np1)(
        b1.reshape(np1 * n, FW), srcl, dstl, cnth)

    h, a2, b2 = _tc_post(x, a1, s1, s2, mn, mx, deg,
                         avgl, post1_W, post1_b, lin1_W, lin1_b,
                         avg_lin, pre_next=(pre2_W, pre2_b))

    f2 = pre2_W.shape[1]
    np2 = f2 // FW
    s1b, s2b, mnb, mxb = _acc_kernel(e, np2)(b2, srcl, dstl, cnth)

    (out,) = _tc_post(h, a2, s1b, s2b, mnb, mxb, deg,
                      avgl, post2_W, post2_b, lin2_W, lin2_b, avg_lin)
    return out


# speedup vs baseline: 4.5430x; 1.0075x over previous
import functools

import jax
import jax.numpy as jnp
from jax import lax
from jax.experimental import pallas as pl
from jax.experimental.pallas import tpu as pltpu
from jax.experimental.pallas import tpu_sc as plsc

NT = 32
R = 313
RT = R + 1
NPADR = NT * R
DEGW = 320
CH = 4000
FLUSH = 2048
C = 256
CG = 128
FW = 64


def _wid():
    return lax.axis_index("s") * 2 + lax.axis_index("c")


@functools.cache
def _bucket_kernel(E):
    EPAD = E + C
    mesh = plsc.VectorSubcoreMesh(core_axis_name="c", subcore_axis_name="s")
    out_type = (
        jax.ShapeDtypeStruct((NT, EPAD), jnp.int32),
        jax.ShapeDtypeStruct((NT, EPAD), jnp.int32),
        jax.ShapeDtypeStruct((NT, 16), jnp.int32),
        jax.ShapeDtypeStruct((NT, DEGW), jnp.float32),
    )
    scratch = [
        pltpu.VMEM((CH,), jnp.int32),
        pltpu.VMEM((CH,), jnp.int32),
        pltpu.VMEM((CH,), jnp.int32),
        pltpu.VMEM((CH,), jnp.int32),
        pltpu.VMEM((FLUSH + 2 * C,), jnp.int32),
        pltpu.VMEM((FLUSH + 2 * C,), jnp.int32),
        pltpu.VMEM((DEGW,), jnp.float32),
        pltpu.VMEM((16,), jnp.int32),
        pltpu.SemaphoreType.DMA,
        pltpu.SemaphoreType.DMA,
    ]

    @functools.partial(pl.kernel, out_type=out_type, mesh=mesh,
                       scratch_types=scratch,
                       compiler_params=pltpu.CompilerParams(
                           use_tc_tiling_on_sc=False,
                           needs_layout_passes=False))
    def bucket(src_h, dst_h, srcl_h, dstl_h, cnt_h, deg_h,
               dbufA, sbufA, dbufB, sbufB, csrc, cdst, degv, cntv,
               semA, semB):
        wid = _wid()
        lo = wid * R

        def zb(i, carry):
            degv[pl.ds(i * 16, 16)] = jnp.zeros((16,), jnp.float32)
            return carry
        lax.fori_loop(0, DEGW // 16, zb, 0)

        ones = jnp.ones((16,), jnp.float32)

        GV = 5

        def make_inner(dbufX, sbufX):
            def inner(j, carry):
                wv, base = carry
                for u in range(GV):
                    d = dbufX[pl.ds((j * GV + u) * 16, 16)]
                    s = sbufX[pl.ds((j * GV + u) * 16, 16)]
                    dl = d - lo
                    m = (dl >= 0) & (dl < R)
                    dlc = jnp.where(m, dl, R)
                    plsc.addupdate_scatter(degv, [dlc], ones, mask=m)
                    pos = wv + plsc.cumsum(jnp.where(m, 1, 0)) - 1
                    plsc.store_scatter(csrc, [pos], s, mask=m)
                    plsc.store_scatter(cdst, [pos], dlc, mask=m)
                    wv = wv + plsc.all_reduce_population_count(m)
                w = wv[0]
                fl = w >= FLUSH

                @pl.when(fl)
                def _():
                    fb = pl.multiple_of(base, FLUSH)
                    pltpu.sync_copy(csrc.at[pl.ds(0, FLUSH)],
                                    srcl_h.at[wid, pl.ds(fb, FLUSH)])
                    pltpu.sync_copy(cdst.at[pl.ds(0, FLUSH)],
                                    dstl_h.at[wid, pl.ds(fb, FLUSH)])
                    for u in range(GV):
                        rs = csrc[pl.ds(FLUSH + u * 16, 16)]
                        csrc[pl.ds(u * 16, 16)] = rs
                        rd = cdst[pl.ds(FLUSH + u * 16, 16)]
                        cdst[pl.ds(u * 16, 16)] = rd

                wv = jnp.where(fl, wv - FLUSH, wv)
                base = jnp.where(fl, base + FLUSH, base)
                return (wv, base)
            return inner

        def edge_refs(i, dbufX, sbufX):
            ib = pl.multiple_of(i * CH, 8)
            return ((dst_h.at[pl.ds(ib, CH)], dbufX),
                    (src_h.at[pl.ds(ib, CH)], sbufX))

        def fire_edges(i, dbufX, sbufX, semX):
            for src, dst in edge_refs(i, dbufX, sbufX):
                pltpu.async_copy(src, dst, semX)

        def wait_edges(i, dbufX, sbufX, semX):
            for src, dst in edge_refs(i, dbufX, sbufX):
                pltpu.make_async_copy(src, dst, semX).wait()

        NCH = E // CH
        for src, dst in edge_refs(0, dbufA, sbufA):
            pltpu.sync_copy(src, dst)
        fire_edges(1, dbufB, sbufB, semB)

        NGR = CH // (16 * GV)

        def outer(p, carry):
            iA = 2 * p
            carry = lax.fori_loop(0, NGR, make_inner(dbufA, sbufA), carry)

            @pl.when(iA + 2 < NCH)
            def _():
                fire_edges(iA + 2, dbufA, sbufA, semA)
            wait_edges(iA + 1, dbufB, sbufB, semB)
            carry = lax.fori_loop(0, NGR, make_inner(dbufB, sbufB), carry)

            @pl.when(iA + 3 < NCH)
            def _():
                fire_edges(iA + 3, dbufB, sbufB, semB)

            @pl.when(iA + 2 < NCH)
            def _():
                wait_edges(iA + 2, dbufA, sbufA, semA)
            return carry

        wv, base = lax.fori_loop(0, NCH // 2, outer,
                                 (jnp.zeros((16,), jnp.int32),
                                  jnp.int32(0)))
        w = wv[0]

        wp = ((w + (C - 1)) // C) * C

        def padb(k, carry):
            csrc[pl.ds(w + k * 16, 16)] = jnp.zeros((16,), jnp.int32)
            cdst[pl.ds(w + k * 16, 16)] = jnp.full((16,), R, jnp.int32)
            return carry
        lax.fori_loop(0, (wp - w + 15) // 16, padb, 0)

        def drain(k, carry):
            db = pl.multiple_of(base + k * C, C)
            pltpu.sync_copy(csrc.at[pl.ds(k * C, C)],
                            srcl_h.at[wid, pl.ds(db, C)])
            pltpu.sync_copy(cdst.at[pl.ds(k * C, C)],
                            dstl_h.at[wid, pl.ds(db, C)])
            return carry
        lax.fori_loop(0, wp // C, drain, 0)

        cntv[...] = jnp.full((16,), base + wp, jnp.int32)
        pltpu.sync_copy(cntv, cnt_h.at[wid])
        pltpu.sync_copy(degv, deg_h.at[wid])

    return bucket


@functools.cache
def _acc_kernel(E, npass):
    K = FW // 16
    mesh = plsc.VectorSubcoreMesh(core_axis_name="c", subcore_axis_name="s")
    out_type = tuple(
        jax.ShapeDtypeStruct((npass, NPADR, FW), jnp.float32)
        for _ in range(4))
    scratch = [
        pltpu.VMEM_SHARED((16 * RT, FW), jnp.float32),
        pltpu.VMEM((RT, FW), jnp.float32),
        [pltpu.VMEM((RT * 16,), jnp.float32) for _ in range(K)],
        [pltpu.VMEM((RT * 16,), jnp.float32) for _ in range(K)],
        pltpu.VMEM((R, FW), jnp.float32),
        pltpu.VMEM((CG, FW), jnp.float32),
        pltpu.VMEM((CG, FW), jnp.float32),
        pltpu.VMEM((CG,), jnp.int32),
        pltpu.VMEM((CG,), jnp.int32),
        pltpu.VMEM((CG,), jnp.int32),
        pltpu.VMEM((CG,), jnp.int32),
        pltpu.VMEM((CG,), jnp.int32),
        pltpu.VMEM((CG,), jnp.int32),
        pltpu.VMEM((CG,), jnp.int32),
        pltpu.VMEM((CG,), jnp.int32),
        pltpu.VMEM((CG + 16,), jnp.int32),
        pltpu.VMEM((16,), jnp.int32),
        pltpu.SemaphoreType.DMA,
        pltpu.SemaphoreType.DMA,
        pltpu.SemaphoreType.DMA,
        pltpu.SemaphoreType.DMA,
        pltpu.SemaphoreType.DMA,
        pltpu.SemaphoreType.DMA,
    ]

    @functools.partial(pl.kernel, out_type=out_type, mesh=mesh,
                       scratch_types=scratch,
                       compiler_params=pltpu.CompilerParams(
                           use_tc_tiling_on_sc=False,
                           needs_layout_passes=False))
    def acc(table_h, srcl_h, dstl_h, cnt_h,
            s1_h, s2_h, mn_h, mx_h,
            spmS, accQ, mnk, mxk, merge, rowsA, rowsB,
            sbufA, sbufB, dbufA, dbufB, giA, giB, sxA, sxB, dacc, cntv,
            semLA, semLB, semGA, semGB, semSA, semSB):
        wid = _wid()
        lo = wid * R
        lbase = lax.axis_index("s") * RT
        pltpu.sync_copy(cnt_h.at[wid], cntv)
        cnt = jnp.max(cntv[...])
        nch = cnt // CG

        def list_refs(c_idx, sbufX, dbufX):
            eb = pl.multiple_of(c_idx * CG, CG)
            return ((srcl_h.at[wid, pl.ds(eb, CG)], sbufX),
                    (dstl_h.at[wid, pl.ds(eb, CG)], dbufX))

        def fire_list(c_idx, sbufX, dbufX, semX):
            for src, dst in list_refs(c_idx, sbufX, dbufX):
                pltpu.async_copy(src, dst, semX)

        def wait_list(c_idx, sbufX, dbufX, semX):
            for src, dst in list_refs(c_idx, sbufX, dbufX):
                pltpu.make_async_copy(src, dst, semX).wait()

        for f in range(npass):
            def fire_gather(sbufX, giX, rowsX, semX):
                for k in range(CG // 16):
                    giX[pl.ds(k * 16, 16)] = \
                        sbufX[pl.ds(k * 16, 16)] * npass + f
                pltpu.async_copy(table_h.at[giX], rowsX, semX)

            def wait_gather(giX, rowsX, semX):
                pltpu.make_async_copy(table_h.at[giX], rowsX, semX).wait()

            def fire_scatter(rowsX, sxX, semX):
                pltpu.async_copy(rowsX, spmS.at[sxX], semX, add=True)

            def wait_scatter(rowsX, sxX, semX):
                pltpu.make_async_copy(rowsX, spmS.at[sxX], semX).wait()

            def do_minmax(rowsX):
                dv0 = dacc[pl.ds(0, 16)]
                offs0 = tuple(dv0[l] for l in range(16))

                def edge(g, offs):
                    dvn = dacc[pl.ds(g * 16 + 16, 16)]
                    noffs = tuple(dvn[l] for l in range(16))
                    for l in range(16):
                        dl = offs[l]
                        off16 = dl * 16
                        e = g * 16 + l
                        for k in range(K):
                            r = rowsX[e, pl.ds(k * 16, 16)]
                            plsc.addupdate(
                                accQ.at[dl, pl.ds(k * 16, 16)], r * r)
                            cm = mnk[k][pl.ds(off16, 16)]
                            mnk[k][pl.ds(off16, 16)] = jnp.minimum(cm, r)
                            cx = mxk[k][pl.ds(off16, 16)]
                            mxk[k][pl.ds(off16, 16)] = jnp.maximum(cx, r)
                    return noffs
                lax.fori_loop(0, CG // 16, edge, offs0)

            def copy_dst(dbufX, sxX):
                for k in range(CG // 16):
                    dlv = dbufX[pl.ds(k * 16, 16)]
                    dacc[pl.ds(k * 16, 16)] = dlv
                    sxX[pl.ds(k * 16, 16)] = dlv + lbase

            def zm(i, carry):
                for k in range(K):
                    z = jnp.zeros((16,), jnp.float32)
                    merge[i, pl.ds(k * 16, 16)] = z
                    accQ[i, pl.ds(k * 16, 16)] = z
                return carry
            lax.fori_loop(0, R, zm, 0)
            for k in range(K):
                accQ[R, pl.ds(k * 16, 16)] = jnp.zeros((16,), jnp.float32)
            pltpu.sync_copy(merge, spmS.at[pl.ds(lbase, R)])
            pltpu.sync_copy(merge.at[pl.ds(0, 1)],
                            spmS.at[pl.ds(lbase + R, 1)])

            def zk(i, carry):
                for k in range(K):
                    mnk[k][pl.ds(i * 16, 16)] = jnp.full((16,), 3e38,
                                                         jnp.float32)
                    mxk[k][pl.ds(i * 16, 16)] = jnp.full((16,), -3e38,
                                                         jnp.float32)
                return carry
            lax.fori_loop(0, RT, zk, 0)

            @pl.when(nch > 0)
            def _():
                for src, dst in list_refs(0, sbufA, dbufA):
                    pltpu.sync_copy(src, dst)
                fire_gather(sbufA, giA, rowsA, semGA)

            @pl.when(nch > 1)
            def _():
                fire_list(1, sbufB, dbufB, semLB)

            def pair(p, carry):
                cA = 2 * p
                cB = cA + 1
                copy_dst(dbufA, sxA)

                @pl.when(cB < nch)
                def _():
                    wait_list(cB, sbufB, dbufB, semLB)
                    fire_gather(sbufB, giB, rowsB, semGB)

                @pl.when(cA + 2 < nch)
                def _():
                    fire_list(cA + 2, sbufA, dbufA, semLA)

                wait_gather(giA, rowsA, semGA)
                fire_scatter(rowsA, sxA, semSA)
                do_minmax(rowsA)
                wait_scatter(rowsA, sxA, semSA)

                @pl.when(cB < nch)
                def _():
                    copy_dst(dbufB, sxB)

                    @pl.when(cB + 2 < nch)
                    def _():
                        fire_list(cB + 2, sbufB, dbufB, semLB)

                    wait_gather(giB, rowsB, semGB)
                    fire_scatter(rowsB, sxB, semSB)
                    do_minmax(rowsB)
                    wait_scatter(rowsB, sxB, semSB)

                    @pl.when(cA + 2 < nch)
                    def _():
                        wait_list(cA + 2, sbufA, dbufA, semLA)
                        fire_gather(sbufA, giA, rowsA, semGA)

                return carry
            lax.fori_loop(0, (nch + 1) // 2, pair, 0)

            pltpu.sync_copy(spmS.at[pl.ds(lbase, R)],
                            s1_h.at[f, pl.ds(lo, R)])
            pltpu.sync_copy(accQ.at[pl.ds(0, R)],
                            s2_h.at[f, pl.ds(lo, R)])
            for kref, out_h in ((mnk, mn_h), (mxk, mx_h)):
                def mg(i, carry):
                    for k in range(K):
                        merge[i, pl.ds(k * 16, 16)] = \
                            kref[k][pl.ds(i * 16, 16)]
                    return carry
                lax.fori_loop(0, R, mg, 0)
                pltpu.sync_copy(merge, out_h.at[f, pl.ds(lo, R)])

    return acc


def _tc_pre(x, Wd, Ws, b, blk=2000):
    n, fin = x.shape
    fo = Wd.shape[1]

    def body(xr, wdr, wsr, br, ar, btr):
        xb = xr[...]
        ar[...] = jnp.dot(xb, wdr[...],
                          preferred_element_type=jnp.float32) + br[...]
        btr[...] = jnp.dot(xb, wsr[...], preferred_element_type=jnp.float32)

    return pl.pallas_call(
        body,
        grid=(n // blk,),
        in_specs=[
            pl.BlockSpec((blk, fin), lambda i: (i, 0)),
            pl.BlockSpec((fin, fo), lambda i: (0, 0)),
            pl.BlockSpec((fin, fo), lambda i: (0, 0)),
            pl.BlockSpec((1, fo), lambda i: (0, 0)),
        ],
        out_specs=[
            pl.BlockSpec((blk, fo), lambda i: (i, 0)),
            pl.BlockSpec((blk, fo), lambda i: (i, 0)),
        ],
        out_shape=[jax.ShapeDtypeStruct((n, fo), jnp.float32)] * 2,
    )(x, Wd, Ws, b.reshape(1, fo))


def _tc_avglog(degp, n):
    def body(degr, outr):
        d = degr[...]
        col = lax.broadcasted_iota(jnp.int32, (NT, DEGW), 1)
        row = lax.broadcasted_iota(jnp.int32, (NT, DEGW), 0)
        valid = (col < R) & (row * R + col < n)
        outr[0, 0] = jnp.sum(jnp.where(valid, jnp.log(d + 1.0), 0.0)) / n

    return pl.pallas_call(
        body,
        in_specs=[pl.BlockSpec((NT, DEGW), lambda: (0, 0))],
        out_specs=pl.BlockSpec(memory_space=pltpu.SMEM),
        out_shape=jax.ShapeDtypeStruct((1, 1), jnp.float32),
    )(degp)


def _tc_post(xin, A, s1, s2, mn, mx, deg, avgl, postW, postb, linW, linb,
             avg_lin, pre_next=None, blk=1000):
    n, fin = xin.shape
    f = A.shape[1]
    npass = s1.shape[0]
    hid = postW.shape[1]
    ho = linW.shape[1]
    wx = postW[:f]
    w1 = postW[f:5 * f]
    w2 = postW[5 * f:9 * f]
    w3 = postW[9 * f:13 * f]
    w4 = postW[13 * f:17 * f]
    fused = pre_next is not None
    if fused:
        pnW, pnb = pre_next
        fn = pnW.shape[1]

    def body(xr, ar, s1r, s2r, mnr, mxr, degr, avr,
             wxr, w1r, w2r, w3r, w4r, pbr, lwr, lbr, *rest):
        deg_b = degr[...]
        dc = jnp.maximum(deg_b, 1.0)
        has = deg_b > 0.0
        cA = ar[...]
        if npass == 2:
            s1v = jnp.concatenate([s1r[0], s1r[1]], axis=-1)
            s2v = jnp.concatenate([s2r[0], s2r[1]], axis=-1)
            mnv = jnp.concatenate([mnr[0], mnr[1]], axis=-1)
            mxv = jnp.concatenate([mxr[0], mxr[1]], axis=-1)
        else:
            s1v, s2v, mnv, mxv = s1r[0], s2r[0], mnr[0], mxr[0]
        s1d = s1v / dc
        mean = jnp.where(has, cA + s1d, 0.0)
        mnx = jnp.where(has, cA + mnv, 0.0)
        mxx = jnp.where(has, cA + mxv, 0.0)
        var = jnp.maximum(s2v / dc - s1d * s1d, 0.0)
        std = jnp.sqrt(var + 1e-5)
        aggr = jnp.concatenate([mean, mnx, mxx, std], axis=-1)
        al = avr[0, 0]
        ld = jnp.log(dc + 1.0)
        o = jnp.dot(xr[...], wxr[...], preferred_element_type=jnp.float32)
        o += jnp.dot(aggr, w1r[...], preferred_element_type=jnp.float32)
        o += jnp.dot(aggr * (ld / al), w2r[...],
                     preferred_element_type=jnp.float32)
        o += jnp.dot(aggr * (al / ld), w3r[...],
                     preferred_element_type=jnp.float32)
        o += jnp.dot(aggr * (dc / avg_lin), w4r[...],
                     preferred_element_type=jnp.float32)
        o += pbr[...]
        o = jnp.dot(o, lwr[...], preferred_element_type=jnp.float32)
        o += lbr[...]
        if fused:
            wdr, wsr, pnbr, hr, a2r, b2r = rest
            h = jnp.maximum(o, 0.0)
            hr[...] = h
            a2r[...] = jnp.dot(h, wdr[...],
                               preferred_element_type=jnp.float32) + pnbr[...]
            b2r[...] = jnp.dot(h, wsr[...],
                               preferred_element_type=jnp.float32)
        else:
            outr, = rest
            om = o - jnp.max(o, axis=1, keepdims=True)
            outr[...] = om - jnp.log(
                jnp.sum(jnp.exp(om), axis=1, keepdims=True))

    def full(a):
        return pl.BlockSpec(a.shape, lambda i: (0,) * a.ndim)

    in_specs = [
        pl.BlockSpec((blk, fin), lambda i: (i, 0)),
        pl.BlockSpec((blk, f), lambda i: (i, 0)),
    ] + [pl.BlockSpec((npass, blk, FW), lambda i: (0, i, 0))] * 4 + [
        pl.BlockSpec((blk, 1), lambda i: (i, 0)),
        pl.BlockSpec(memory_space=pltpu.SMEM),
        full(wx), full(w1), full(w2), full(w3), full(w4),
        pl.BlockSpec((1, hid), lambda i: (0, 0)),
        full(linW),
        pl.BlockSpec((1, ho), lambda i: (0, 0)),
    ]
    args = [xin, A, s1, s2, mn, mx, deg, avgl,
            wx, w1, w2, w3, w4, postb.reshape(1, hid), linW,
            linb.reshape(1, ho)]
    if fused:
        in_specs += [full(pnW[:hid]), full(pnW[hid:]),
                     pl.BlockSpec((1, fn), lambda i: (0, 0))]
        args += [pnW[:hid], pnW[hid:], pnb.reshape(1, fn)]
        out_specs = [pl.BlockSpec((blk, ho), lambda i: (i, 0)),
                     pl.BlockSpec((blk, fn), lambda i: (i, 0)),
                     pl.BlockSpec((blk, fn), lambda i: (i, 0))]
        out_shape = [jax.ShapeDtypeStruct((n, ho), jnp.float32),
                     jax.ShapeDtypeStruct((n, fn), jnp.float32),
                     jax.ShapeDtypeStruct((n, fn), jnp.float32)]
    else:
        out_specs = [pl.BlockSpec((blk, ho), lambda i: (i, 0))]
        out_shape = [jax.ShapeDtypeStruct((n, ho), jnp.float32)]

    return pl.pallas_call(
        body,
        grid=(n // blk,),
        in_specs=in_specs,
        out_specs=out_specs,
        out_shape=out_shape,
    )(*args)


def kernel(x, edge_index, pre1_W, pre1_b, post1_W, post1_b, lin1_W, lin1_b,
           pre2_W, pre2_b, post2_W, post2_b, lin2_W, lin2_b):
    n, f_in = x.shape
    e = edge_index.shape[1]
    f1 = pre1_W.shape[1]
    avg_lin = float(e) / float(n)

    srcl, dstl, cnth, degp = _bucket_kernel(e)(edge_index[0],
                                               edge_index[1])
    deg = degp[:, :R].reshape(-1)[:n].reshape(n, 1)
    avgl = _tc_avglog(degp, n)

    a1, b1 = _tc_pre(x, pre1_W[:f_in], pre1_W[f_in:], pre1_b)
    np1 = f1 // FW
    s1, s2, mn, mx = _acc_kernel(e, np1)(
        b1.reshape(np1 * n, FW), srcl, dstl, cnth)

    h, a2, b2 = _tc_post(x, a1, s1, s2, mn, mx, deg,
                         avgl, post1_W, post1_b, lin1_W, lin1_b,
                         avg_lin, pre_next=(pre2_W, pre2_b))

    f2 = pre2_W.shape[1]
    np2 = f2 // FW
    s1b, s2b, mnb, mxb = _acc_kernel(e, np2)(b2, srcl, dstl, cnth)

    (out,) = _tc_post(h, a2, s1b, s2b, mnb, mxb, deg,
                      avgl, post2_W, post2_b, lin2_W, lin2_b, avg_lin)
    return out


# speedup vs baseline: 4.5910x; 1.0106x over previous
import functools

import jax
import jax.numpy as jnp
from jax import lax
from jax.experimental import pallas as pl
from jax.experimental.pallas import tpu as pltpu
from jax.experimental.pallas import tpu_sc as plsc

NT = 32
R = 313
RT = R + 1
NPADR = NT * R
DEGW = 320
CH = 4000
FLUSH = 2048
C = 256
CG = 128
FW = 64


def _wid():
    return lax.axis_index("s") * 2 + lax.axis_index("c")


@functools.cache
def _bucket_kernel(E):
    EPAD = E + C
    mesh = plsc.VectorSubcoreMesh(core_axis_name="c", subcore_axis_name="s")
    out_type = (
        jax.ShapeDtypeStruct((NT, EPAD), jnp.int32),
        jax.ShapeDtypeStruct((NT, EPAD), jnp.int32),
        jax.ShapeDtypeStruct((NT, 16), jnp.int32),
        jax.ShapeDtypeStruct((NT, DEGW), jnp.float32),
    )
    scratch = [
        pltpu.VMEM((CH,), jnp.int32),
        pltpu.VMEM((CH,), jnp.int32),
        pltpu.VMEM((CH,), jnp.int32),
        pltpu.VMEM((CH,), jnp.int32),
        pltpu.VMEM((FLUSH + 2 * C,), jnp.int32),
        pltpu.VMEM((FLUSH + 2 * C,), jnp.int32),
        pltpu.VMEM((DEGW,), jnp.float32),
        pltpu.VMEM((16,), jnp.int32),
        pltpu.SemaphoreType.DMA,
        pltpu.SemaphoreType.DMA,
    ]

    @functools.partial(pl.kernel, out_type=out_type, mesh=mesh,
                       scratch_types=scratch,
                       compiler_params=pltpu.CompilerParams(
                           use_tc_tiling_on_sc=False,
                           needs_layout_passes=False))
    def bucket(src_h, dst_h, srcl_h, dstl_h, cnt_h, deg_h,
               dbufA, sbufA, dbufB, sbufB, csrc, cdst, degv, cntv,
               semA, semB):
        wid = _wid()
        lo = wid * R

        def zb(i, carry):
            degv[pl.ds(i * 16, 16)] = jnp.zeros((16,), jnp.float32)
            return carry
        lax.fori_loop(0, DEGW // 16, zb, 0)

        ones = jnp.ones((16,), jnp.float32)

        GV = 5

        def make_inner(dbufX, sbufX):
            def inner(j, carry):
                wv, base = carry
                for u in range(GV):
                    d = dbufX[pl.ds((j * GV + u) * 16, 16)]
                    s = sbufX[pl.ds((j * GV + u) * 16, 16)]
                    dl = d - lo
                    m = (dl >= 0) & (dl < R)
                    dlc = jnp.where(m, dl, R)
                    plsc.addupdate_scatter(degv, [dlc], ones, mask=m)
                    pos = wv + plsc.cumsum(jnp.where(m, 1, 0)) - 1
                    plsc.store_scatter(csrc, [pos], s, mask=m)
                    plsc.store_scatter(cdst, [pos], dlc, mask=m)
                    wv = wv + plsc.all_reduce_population_count(m)
                w = wv[0]
                fl = w >= FLUSH

                @pl.when(fl)
                def _():
                    fb = pl.multiple_of(base, FLUSH)
                    pltpu.sync_copy(csrc.at[pl.ds(0, FLUSH)],
                                    srcl_h.at[wid, pl.ds(fb, FLUSH)])
                    pltpu.sync_copy(cdst.at[pl.ds(0, FLUSH)],
                                    dstl_h.at[wid, pl.ds(fb, FLUSH)])
                    for u in range(GV):
                        rs = csrc[pl.ds(FLUSH + u * 16, 16)]
                        csrc[pl.ds(u * 16, 16)] = rs
                        rd = cdst[pl.ds(FLUSH + u * 16, 16)]
                        cdst[pl.ds(u * 16, 16)] = rd

                wv = jnp.where(fl, wv - FLUSH, wv)
                base = jnp.where(fl, base + FLUSH, base)
                return (wv, base)
            return inner

        def edge_refs(i, dbufX, sbufX):
            ib = pl.multiple_of(i * CH, 8)
            return ((dst_h.at[pl.ds(ib, CH)], dbufX),
                    (src_h.at[pl.ds(ib, CH)], sbufX))

        def fire_edges(i, dbufX, sbufX, semX):
            for src, dst in edge_refs(i, dbufX, sbufX):
                pltpu.async_copy(src, dst, semX)

        def wait_edges(i, dbufX, sbufX, semX):
            for src, dst in edge_refs(i, dbufX, sbufX):
                pltpu.make_async_copy(src, dst, semX).wait()

        NCH = E // CH
        for src, dst in edge_refs(0, dbufA, sbufA):
            pltpu.sync_copy(src, dst)
        fire_edges(1, dbufB, sbufB, semB)

        NGR = CH // (16 * GV)

        def outer(p, carry):
            iA = 2 * p
            carry = lax.fori_loop(0, NGR, make_inner(dbufA, sbufA), carry)

            @pl.when(iA + 2 < NCH)
            def _():
                fire_edges(iA + 2, dbufA, sbufA, semA)
            wait_edges(iA + 1, dbufB, sbufB, semB)
            carry = lax.fori_loop(0, NGR, make_inner(dbufB, sbufB), carry)

            @pl.when(iA + 3 < NCH)
            def _():
                fire_edges(iA + 3, dbufB, sbufB, semB)

            @pl.when(iA + 2 < NCH)
            def _():
                wait_edges(iA + 2, dbufA, sbufA, semA)
            return carry

        wv, base = lax.fori_loop(0, NCH // 2, outer,
                                 (jnp.zeros((16,), jnp.int32),
                                  jnp.int32(0)))
        w = wv[0]

        wp = ((w + (C - 1)) // C) * C

        def padb(k, carry):
            csrc[pl.ds(w + k * 16, 16)] = jnp.zeros((16,), jnp.int32)
            cdst[pl.ds(w + k * 16, 16)] = jnp.full((16,), R, jnp.int32)
            return carry
        lax.fori_loop(0, (wp - w + 15) // 16, padb, 0)

        def drain(k, carry):
            db = pl.multiple_of(base + k * C, C)
            pltpu.sync_copy(csrc.at[pl.ds(k * C, C)],
                            srcl_h.at[wid, pl.ds(db, C)])
            pltpu.sync_copy(cdst.at[pl.ds(k * C, C)],
                            dstl_h.at[wid, pl.ds(db, C)])
            return carry
        lax.fori_loop(0, wp // C, drain, 0)

        cntv[...] = jnp.full((16,), base + wp, jnp.int32)
        pltpu.sync_copy(cntv, cnt_h.at[wid])
        pltpu.sync_copy(degv, deg_h.at[wid])

    return bucket


@functools.cache
def _acc_kernel(E, npass):
    K = FW // 16
    CB = 2 * CG
    mesh = plsc.VectorSubcoreMesh(core_axis_name="c", subcore_axis_name="s")
    out_type = tuple(
        jax.ShapeDtypeStruct((npass, NPADR, FW), jnp.float32)
        for _ in range(4))
    scratch = [
        pltpu.VMEM_SHARED((16 * RT, FW), jnp.float32),
        pltpu.VMEM((RT, FW), jnp.float32),
        [pltpu.VMEM((RT * 16,), jnp.float32) for _ in range(K)],
        [pltpu.VMEM((RT * 16,), jnp.float32) for _ in range(K)],
        pltpu.VMEM((CB, FW), jnp.float32),
        pltpu.VMEM((CB, FW), jnp.float32),
        pltpu.VMEM((CB,), jnp.int32),
        pltpu.VMEM((CB,), jnp.int32),
        pltpu.VMEM((CB,), jnp.int32),
        pltpu.VMEM((CB,), jnp.int32),
        pltpu.VMEM((2, CG), jnp.int32),
        pltpu.VMEM((2, CG), jnp.int32),
        pltpu.VMEM((2, CG), jnp.int32),
        pltpu.VMEM((2, CG), jnp.int32),
        pltpu.VMEM((CB + 16,), jnp.int32),
        pltpu.VMEM((16,), jnp.int32),
        pltpu.SemaphoreType.DMA,
        pltpu.SemaphoreType.DMA,
        pltpu.SemaphoreType.DMA,
        pltpu.SemaphoreType.DMA,
        pltpu.SemaphoreType.DMA,
        pltpu.SemaphoreType.DMA,
    ]

    @functools.partial(pl.kernel, out_type=out_type, mesh=mesh,
                       scratch_types=scratch,
                       compiler_params=pltpu.CompilerParams(
                           use_tc_tiling_on_sc=False,
                           needs_layout_passes=False))
    def acc(table_h, srcl_h, dstl_h, cnt_h,
            s1_h, s2_h, mn_h, mx_h,
            spmS, accQ, mnk, mxk, rowsA, rowsB,
            sbufA, sbufB, dbufA, dbufB, giA, giB, sxA, sxB, dacc, cntv,
            semLA, semLB, semGA, semGB, semSA, semSB):
        wid = _wid()
        lo = wid * R
        lbase = lax.axis_index("s") * RT
        pltpu.sync_copy(cnt_h.at[wid], cntv)
        cnt = jnp.max(cntv[...])
        nch = cnt // CB

        def list_refs(c_idx, sbufX, dbufX):
            eb = pl.multiple_of(c_idx * CB, CB)
            return ((srcl_h.at[wid, pl.ds(eb, CB)], sbufX),
                    (dstl_h.at[wid, pl.ds(eb, CB)], dbufX))

        def fire_list(c_idx, sbufX, dbufX, semX):
            for src, dst in list_refs(c_idx, sbufX, dbufX):
                pltpu.async_copy(src, dst, semX)

        def wait_list(c_idx, sbufX, dbufX, semX):
            for src, dst in list_refs(c_idx, sbufX, dbufX):
                pltpu.make_async_copy(src, dst, semX).wait()

        for f in range(npass):
            def fire_gather(sbufX, giX, rowsX, semX):
                for k in range(CB // 16):
                    giX[k * 16 // CG, pl.ds((k * 16) % CG, 16)] = \
                        sbufX[pl.ds(k * 16, 16)] * npass + f
                for p in range(2):
                    pltpu.async_copy(table_h.at[giX.at[p]],
                                     rowsX.at[pl.ds(p * CG, CG)], semX)

            def wait_gather(giX, rowsX, semX):
                for p in range(2):
                    pltpu.make_async_copy(
                        table_h.at[giX.at[p]],
                        rowsX.at[pl.ds(p * CG, CG)], semX).wait()

            def fire_scatter(rowsX, sxX, semX):
                for p in range(2):
                    pltpu.async_copy(rowsX.at[pl.ds(p * CG, CG)],
                                     spmS.at[sxX.at[p]], semX, add=True)

            def wait_scatter(rowsX, sxX, semX):
                for p in range(2):
                    pltpu.make_async_copy(
                        rowsX.at[pl.ds(p * CG, CG)],
                        spmS.at[sxX.at[p]], semX).wait()

            def do_minmax(rowsX):
                dv0 = dacc[pl.ds(0, 16)]
                offs0 = tuple(dv0[l] for l in range(16))

                def edge(g, offs):
                    dvn = dacc[pl.ds(g * 16 + 16, 16)]
                    noffs = tuple(dvn[l] for l in range(16))
                    for l in range(16):
                        dl = offs[l]
                        off16 = dl * 16
                        e = g * 16 + l
                        for k in range(K):
                            r = rowsX[e, pl.ds(k * 16, 16)]
                            plsc.addupdate(
                                accQ.at[dl, pl.ds(k * 16, 16)], r * r)
                            cm = mnk[k][pl.ds(off16, 16)]
                            mnk[k][pl.ds(off16, 16)] = jnp.minimum(cm, r)
                            cx = mxk[k][pl.ds(off16, 16)]
                            mxk[k][pl.ds(off16, 16)] = jnp.maximum(cx, r)
                    return noffs
                lax.fori_loop(0, CB // 16, edge, offs0)

            def copy_dst(dbufX, sxX):
                for k in range(CB // 16):
                    dlv = dbufX[pl.ds(k * 16, 16)]
                    dacc[pl.ds(k * 16, 16)] = dlv
                    sxX[k * 16 // CG, pl.ds((k * 16) % CG, 16)] = \
                        dlv + lbase

            def zm(i, carry):
                for k in range(K):
                    accQ[i, pl.ds(k * 16, 16)] = jnp.zeros((16,),
                                                           jnp.float32)
                return carry
            lax.fori_loop(0, RT, zm, 0)
            pltpu.sync_copy(accQ, spmS.at[pl.ds(lbase, RT)])

            def zk(i, carry):
                for k in range(K):
                    mnk[k][pl.ds(i * 16, 16)] = jnp.full((16,), 3e38,
                                                         jnp.float32)
                    mxk[k][pl.ds(i * 16, 16)] = jnp.full((16,), -3e38,
                                                         jnp.float32)
                return carry
            lax.fori_loop(0, RT, zk, 0)

            @pl.when(nch > 0)
            def _():
                for src, dst in list_refs(0, sbufA, dbufA):
                    pltpu.sync_copy(src, dst)
                fire_gather(sbufA, giA, rowsA, semGA)

            @pl.when(nch > 1)
            def _():
                fire_list(1, sbufB, dbufB, semLB)

            def pair(p, carry):
                cA = 2 * p
                cB = cA + 1
                copy_dst(dbufA, sxA)

                @pl.when(cB < nch)
                def _():
                    wait_list(cB, sbufB, dbufB, semLB)
                    fire_gather(sbufB, giB, rowsB, semGB)

                @pl.when(cA + 2 < nch)
                def _():
                    fire_list(cA + 2, sbufA, dbufA, semLA)

                wait_gather(giA, rowsA, semGA)
                fire_scatter(rowsA, sxA, semSA)
                do_minmax(rowsA)
                wait_scatter(rowsA, sxA, semSA)

                @pl.when(cB < nch)
                def _():
                    copy_dst(dbufB, sxB)

                    @pl.when(cB + 2 < nch)
                    def _():
                        fire_list(cB + 2, sbufB, dbufB, semLB)

                    wait_gather(giB, rowsB, semGB)
                    fire_scatter(rowsB, sxB, semSB)
                    do_minmax(rowsB)
                    wait_scatter(rowsB, sxB, semSB)

                    @pl.when(cA + 2 < nch)
                    def _():
                        wait_list(cA + 2, sbufA, dbufA, semLA)
                        fire_gather(sbufA, giA, rowsA, semGA)

                return carry
            lax.fori_loop(0, (nch + 1) // 2, pair, 0)

            pltpu.sync_copy(spmS.at[pl.ds(lbase, R)],
                            s1_h.at[f, pl.ds(lo, R)])
            pltpu.sync_copy(accQ.at[pl.ds(0, R)],
                            s2_h.at[f, pl.ds(lo, R)])
            for kref, out_h in ((mnk, mn_h), (mxk, mx_h)):
                def mg(i, carry):
                    for k in range(K):
                        accQ[i, pl.ds(k * 16, 16)] = \
                            kref[k][pl.ds(i * 16, 16)]
                    return carry
                lax.fori_loop(0, R, mg, 0)
                pltpu.sync_copy(accQ.at[pl.ds(0, R)],
                                out_h.at[f, pl.ds(lo, R)])

    return acc


def _tc_pre(x, Wd, Ws, b, blk=2000):
    n, fin = x.shape
    fo = Wd.shape[1]

    def body(xr, wdr, wsr, br, ar, btr):
        xb = xr[...]
        ar[...] = jnp.dot(xb, wdr[...],
                          preferred_element_type=jnp.float32) + br[...]
        btr[...] = jnp.dot(xb, wsr[...], preferred_element_type=jnp.float32)

    return pl.pallas_call(
        body,
        grid=(n // blk,),
        in_specs=[
            pl.BlockSpec((blk, fin), lambda i: (i, 0)),
            pl.BlockSpec((fin, fo), lambda i: (0, 0)),
            pl.BlockSpec((fin, fo), lambda i: (0, 0)),
            pl.BlockSpec((1, fo), lambda i: (0, 0)),
        ],
        out_specs=[
            pl.BlockSpec((blk, fo), lambda i: (i, 0)),
            pl.BlockSpec((blk, fo), lambda i: (i, 0)),
        ],
        out_shape=[jax.ShapeDtypeStruct((n, fo), jnp.float32)] * 2,
    )(x, Wd, Ws, b.reshape(1, fo))


def _tc_avglog(degp, n):
    def body(degr, outr):
        d = degr[...]
        col = lax.broadcasted_iota(jnp.int32, (NT, DEGW), 1)
        row = lax.broadcasted_iota(jnp.int32, (NT, DEGW), 0)
        valid = (col < R) & (row * R + col < n)
        outr[0, 0] = jnp.sum(jnp.where(valid, jnp.log(d + 1.0), 0.0)) / n

    return pl.pallas_call(
        body,
        in_specs=[pl.BlockSpec((NT, DEGW), lambda: (0, 0))],
        out_specs=pl.BlockSpec(memory_space=pltpu.SMEM),
        out_shape=jax.ShapeDtypeStruct((1, 1), jnp.float32),
    )(degp)


def _tc_post(xin, A, s1, s2, mn, mx, deg, avgl, postW, postb, linW, linb,
             avg_lin, pre_next=None, blk=1000):
    n, fin = xin.shape
    f = A.shape[1]
    npass = s1.shape[0]
    hid = postW.shape[1]
    ho = linW.shape[1]
    wx = postW[:f]
    w1 = postW[f:5 * f]
    w2 = postW[5 * f:9 * f]
    w3 = postW[9 * f:13 * f]
    w4 = postW[13 * f:17 * f]
    fused = pre_next is not None
    if fused:
        pnW, pnb = pre_next
        fn = pnW.shape[1]

    def body(xr, ar, s1r, s2r, mnr, mxr, degr, avr,
             wxr, w1r, w2r, w3r, w4r, pbr, lwr, lbr, *rest):
        deg_b = degr[...]
        dc = jnp.maximum(deg_b, 1.0)
        has = deg_b > 0.0
        cA = ar[...]
        if npass == 2:
            s1v = jnp.concatenate([s1r[0], s1r[1]], axis=-1)
            s2v = jnp.concatenate([s2r[0], s2r[1]], axis=-1)
            mnv = jnp.concatenate([mnr[0], mnr[1]], axis=-1)
            mxv = jnp.concatenate([mxr[0], mxr[1]], axis=-1)
        else:
            s1v, s2v, mnv, mxv = s1r[0], s2r[0], mnr[0], mxr[0]
        s1d = s1v / dc
        mean = jnp.where(has, cA + s1d, 0.0)
        mnx = jnp.where(has, cA + mnv, 0.0)
        mxx = jnp.where(has, cA + mxv, 0.0)
        var = jnp.maximum(s2v / dc - s1d * s1d, 0.0)
        std = jnp.sqrt(var + 1e-5)
        aggr = jnp.concatenate([mean, mnx, mxx, std], axis=-1)
        al = avr[0, 0]
        ld = jnp.log(dc + 1.0)
        o = jnp.dot(xr[...], wxr[...], preferred_element_type=jnp.float32)
        o += jnp.dot(aggr, w1r[...], preferred_element_type=jnp.float32)
        o += jnp.dot(aggr * (ld / al), w2r[...],
                     preferred_element_type=jnp.float32)
        o += jnp.dot(aggr * (al / ld), w3r[...],
                     preferred_element_type=jnp.float32)
        o += jnp.dot(aggr * (dc / avg_lin), w4r[...],
                     preferred_element_type=jnp.float32)
        o += pbr[...]
        o = jnp.dot(o, lwr[...], preferred_element_type=jnp.float32)
        o += lbr[...]
        if fused:
            wdr, wsr, pnbr, hr, a2r, b2r = rest
            h = jnp.maximum(o, 0.0)
            hr[...] = h
            a2r[...] = jnp.dot(h, wdr[...],
                               preferred_element_type=jnp.float32) + pnbr[...]
            b2r[...] = jnp.dot(h, wsr[...],
                               preferred_element_type=jnp.float32)
        else:
            outr, = rest
            om = o - jnp.max(o, axis=1, keepdims=True)
            outr[...] = om - jnp.log(
                jnp.sum(jnp.exp(om), axis=1, keepdims=True))

    def full(a):
        return pl.BlockSpec(a.shape, lambda i: (0,) * a.ndim)

    in_specs = [
        pl.BlockSpec((blk, fin), lambda i: (i, 0)),
        pl.BlockSpec((blk, f), lambda i: (i, 0)),
    ] + [pl.BlockSpec((npass, blk, FW), lambda i: (0, i, 0))] * 4 + [
        pl.BlockSpec((blk, 1), lambda i: (i, 0)),
        pl.BlockSpec(memory_space=pltpu.SMEM),
        full(wx), full(w1), full(w2), full(w3), full(w4),
        pl.BlockSpec((1, hid), lambda i: (0, 0)),
        full(linW),
        pl.BlockSpec((1, ho), lambda i: (0, 0)),
    ]
    args = [xin, A, s1, s2, mn, mx, deg, avgl,
            wx, w1, w2, w3, w4, postb.reshape(1, hid), linW,
            linb.reshape(1, ho)]
    if fused:
        in_specs += [full(pnW[:hid]), full(pnW[hid:]),
                     pl.BlockSpec((1, fn), lambda i: (0, 0))]
        args += [pnW[:hid], pnW[hid:], pnb.reshape(1, fn)]
        out_specs = [pl.BlockSpec((blk, ho), lambda i: (i, 0)),
                     pl.BlockSpec((blk, fn), lambda i: (i, 0)),
                     pl.BlockSpec((blk, fn), lambda i: (i, 0))]
        out_shape = [jax.ShapeDtypeStruct((n, ho), jnp.float32),
                     jax.ShapeDtypeStruct((n, fn), jnp.float32),
                     jax.ShapeDtypeStruct((n, fn), jnp.float32)]
    else:
        out_specs = [pl.BlockSpec((blk, ho), lambda i: (i, 0))]
        out_shape = [jax.ShapeDtypeStruct((n, ho), jnp.float32)]

    return pl.pallas_call(
        body,
        grid=(n // blk,),
        in_specs=in_specs,
        out_specs=out_specs,
        out_shape=out_shape,
    )(*args)


def kernel(x, edge_index, pre1_W, pre1_b, post1_W, post1_b, lin1_W, lin1_b,
           pre2_W, pre2_b, post2_W, post2_b, lin2_W, lin2_b):
    n, f_in = x.shape
    e = edge_index.shape[1]
    f1 = pre1_W.shape[1]
    avg_lin = float(e) / float(n)

    srcl, dstl, cnth, degp = _bucket_kernel(e)(edge_index[0],
                                               edge_index[1])
    deg = degp[:, :R].reshape(-1)[:n].reshape(n, 1)
    avgl = _tc_avglog(degp, n)

    a1, b1 = _tc_pre(x, pre1_W[:f_in], pre1_W[f_in:], pre1_b)
    np1 = f1 // FW
    s1, s2, mn, mx = _acc_kernel(e, np1)(
        b1.reshape(np1 * n, FW), srcl, dstl, cnth)

    h, a2, b2 = _tc_post(x, a1, s1, s2, mn, mx, deg,
                         avgl, post1_W, post1_b, lin1_W, lin1_b,
                         avg_lin, pre_next=(pre2_W, pre2_b))

    f2 = pre2_W.shape[1]
    np2 = f2 // FW
    s1b, s2b, mnb, mxb = _acc_kernel(e, np2)(b2, srcl, dstl, cnth)

    (out,) = _tc_post(h, a2, s1b, s2b, mnb, mxb, deg,
                      avgl, post2_W, post2_b, lin2_W, lin2_b, avg_lin)
    return out
